# Initial kernel scaffold; baseline (speedup 1.0000x reference)
import jax
import jax.numpy as jnp
from jax.experimental import pallas as pl


def kernel(h, edge_index, coord, edge_attr, W_e1, b_e1, g_e1, be_e1, W_e2, b_e2, g_e2, be_e2, W_n1, b_n1, g_n1, be_n1, W_n2, b_n2, W_c1, b_c1, g_c1, be_c1, W_c2):
    raise NotImplementedError("write your pallas kernel here")


# speedup vs baseline: 1.9511x; 1.9511x over previous
import functools

import jax
import jax.numpy as jnp
from jax import lax
from jax.experimental import pallas as pl
from jax.experimental.pallas import tpu as pltpu
from jax.experimental.pallas import tpu_sc as plsc

NC = 2
NS = 16
NW = NC * NS
CHUNK = 128


def _pick_div(n, cap, mult=1):
    for d in range(min(n, cap), 0, -1):
        if n % d == 0 and d % mult == 0:
            return d
    return 1


def _ln_rows(x, g, b):
    m = jnp.mean(x, axis=-1, keepdims=True)
    v = jnp.mean((x - m) ** 2, axis=-1, keepdims=True)
    return (x - m) / jnp.sqrt(v + 1e-5) * g + b


def _silu(x):
    return x * jax.nn.sigmoid(x)


def kernel(h, edge_index, coord, edge_attr, W_e1, b_e1, g_e1, be_e1, W_e2,
           b_e2, g_e2, be_e2, W_n1, b_n1, g_n1, be_n1, W_n2, b_n2, W_c1,
           b_c1, g_c1, be_c1, W_c2):
    N, H = h.shape
    E = edge_index.shape[1]
    f32 = jnp.float32

    NCH = -(-E // (NW * CHUNK))
    NCH += NCH % 2
    EP = NW * NCH * CHUNK
    EW = EP // NW
    EPT = EP // NS
    NCH4 = EPT // CHUNK
    NP = -(-(N + 1) // 1024) * 1024
    NZT = NP // NS
    ZB = _pick_div(NZT, 128)
    NBLK = _pick_div(NP, 1024, 8)
    NBLK5 = _pick_div(N, 1024, 8)
    EBLK = _pick_div(EP, 2048, 8)

    row1 = lambda a: a.reshape(1, H)

    hp = jnp.pad(h, ((0, NP - N), (0, 0)))
    coordp = jnp.pad(coord, ((0, NP - N), (0, H - coord.shape[1])))
    eip = jnp.concatenate(
        [edge_index, jnp.full((2, EP - E), N, dtype=edge_index.dtype)], axis=1)
    ei32 = eip.reshape(2, NW, NCH, CHUNK)
    row16 = eip[0].reshape(NS, NCH4, CHUNK)
    eap = jnp.pad(edge_attr, ((0, EP - E), (0, 0)))
    w1a = W_e1[0:H]
    w1b = W_e1[H:2 * H]
    w1c = W_e1[2 * H:2 * H + 1]
    w1d = W_e1[2 * H + 1:]
    ED = w1d.shape[0]
    wn1a = W_n1[0:H]
    wn1b = W_n1[H:2 * H]
    wc2r = W_c2.reshape(1, H)

    def pq_body(h_ref, wa_ref, wb_ref, b1_ref, p_ref, q_ref):
        hh = h_ref[...]
        p_ref[...] = jnp.dot(hh, wa_ref[...], preferred_element_type=f32) + b1_ref[...]
        q_ref[...] = jnp.dot(hh, wb_ref[...], preferred_element_type=f32)

    P, Q = pl.pallas_call(
        pq_body,
        grid=(NP // NBLK,),
        in_specs=[
            pl.BlockSpec((NBLK, H), lambda i: (i, 0)),
            pl.BlockSpec((H, H), lambda i: (0, 0)),
            pl.BlockSpec((H, H), lambda i: (0, 0)),
            pl.BlockSpec((1, H), lambda i: (0, 0)),
        ],
        out_specs=[
            pl.BlockSpec((NBLK, H), lambda i: (i, 0)),
            pl.BlockSpec((NBLK, H), lambda i: (i, 0)),
        ],
        out_shape=[
            jax.ShapeDtypeStruct((NP, H), f32),
            jax.ShapeDtypeStruct((NP, H), f32),
        ],
    )(hp, w1a, w1b, row1(b_e1))

    mesh = plsc.VectorSubcoreMesh(core_axis_name="c", subcore_axis_name="s",
                                  num_cores=NC, num_subcores=NS)

    @functools.partial(
        pl.kernel, mesh=mesh,
        out_type=(
            jax.ShapeDtypeStruct((EP, H), f32),
            jax.ShapeDtypeStruct((EP, H), f32),
            jax.ShapeDtypeStruct((EP, H), f32),
            jax.ShapeDtypeStruct((EP, H), f32),
        ),
        scratch_types=[
            pltpu.VMEM((NCH, CHUNK), jnp.int32),
            pltpu.VMEM((NCH, CHUNK), jnp.int32),
            pltpu.VMEM((CHUNK, H), f32),
            pltpu.VMEM((CHUNK, H), f32),
            pltpu.VMEM((CHUNK, H), f32),
            pltpu.VMEM((CHUNK, H), f32),
            pltpu.SemaphoreType.DMA,
        ],
    )
    def gather_k(p_hbm, q_hbm, cp_hbm, ei_hbm, xr_hbm, xc_hbm, cr_hbm, cc_hbm,
                 idxr, idxc, bufp, bufq, bufa, bufb, gsem):
        cid = lax.axis_index("c")
        sid = lax.axis_index("s")
        wid = sid * NC + cid
        base = wid * EW
        pltpu.sync_copy(ei_hbm.at[0, wid], idxr)
        pltpu.sync_copy(ei_hbm.at[1, wid], idxc)

        def chunk(j, carry):
            er = idxr.at[j]
            ec = idxc.at[j]
            c1 = pltpu.async_copy(p_hbm.at[er], bufp, gsem)
            c2 = pltpu.async_copy(q_hbm.at[ec], bufq, gsem)
            c3 = pltpu.async_copy(cp_hbm.at[er], bufa, gsem)
            c4 = pltpu.async_copy(cp_hbm.at[ec], bufb, gsem)
            c1.wait(); c2.wait(); c3.wait(); c4.wait()
            off = base + j * CHUNK
            pltpu.sync_copy(bufp, xr_hbm.at[pl.ds(off, CHUNK)])
            pltpu.sync_copy(bufq, xc_hbm.at[pl.ds(off, CHUNK)])
            pltpu.sync_copy(bufa, cr_hbm.at[pl.ds(off, CHUNK)])
            pltpu.sync_copy(bufb, cc_hbm.at[pl.ds(off, CHUNK)])
            return carry

        lax.fori_loop(0, NCH, chunk, 0)

    XR, XC, CR, CC = gather_k(P, Q, coordp, ei32)

    def edge_body(xr_ref, xc_ref, cr_ref, cc_ref, ea_ref,
                  w1c_ref, w1d_ref, g1_ref, bb1_ref,
                  we2_ref, b2_ref, g2_ref, bb2_ref,
                  wc1_ref, bc1_ref, gc_ref, bbc_ref, wc2_ref,
                  ef_ref, tr_ref):
        cd = cr_ref[...] - cc_ref[...]
        radial = jnp.sum(cd * cd, axis=1, keepdims=True)
        x1 = (xr_ref[...] + xc_ref[...] + radial * w1c_ref[...]
              + jnp.dot(ea_ref[...], w1d_ref[...], preferred_element_type=f32))
        h1 = _silu(_ln_rows(x1, g1_ref[...], bb1_ref[...]))
        x2 = jnp.dot(h1, we2_ref[...], preferred_element_type=f32) + b2_ref[...]
        ef = _silu(_ln_rows(x2, g2_ref[...], bb2_ref[...]))
        ef_ref[...] = ef
        x3 = jnp.dot(ef, wc1_ref[...], preferred_element_type=f32) + bc1_ref[...]
        c1 = _silu(_ln_rows(x3, gc_ref[...], bbc_ref[...]))
        s = jnp.sum(c1 * wc2_ref[...], axis=1, keepdims=True)
        norm = jnp.sqrt(radial + 1e-08)
        tr_ref[...] = cd / (norm + 1.0) * s

    big = lambda: pl.BlockSpec((EBLK, H), lambda i: (i, 0))
    wfull = lambda: pl.BlockSpec((H, H), lambda i: (0, 0))
    prow = lambda: pl.BlockSpec((1, H), lambda i: (0, 0))

    EF, TR = pl.pallas_call(
        edge_body,
        grid=(EP // EBLK,),
        in_specs=[
            big(), big(), big(), big(),
            pl.BlockSpec((EBLK, ED), lambda i: (i, 0)),
            prow(), pl.BlockSpec((ED, H), lambda i: (0, 0)), prow(), prow(),
            wfull(), prow(), prow(), prow(),
            wfull(), prow(), prow(), prow(), prow(),
        ],
        out_specs=[big(), big()],
        out_shape=[
            jax.ShapeDtypeStruct((EP, H), f32),
            jax.ShapeDtypeStruct((EP, H), f32),
        ],
    )(XR, XC, CR, CC, eap,
      w1c, w1d, row1(g_e1), row1(be_e1),
      W_e2, row1(b_e2), row1(g_e2), row1(be_e2),
      W_c1, row1(b_c1), row1(g_c1), row1(be_c1), wc2r)

    @functools.partial(
        pl.kernel, mesh=mesh,
        out_type=(
            jax.ShapeDtypeStruct((NP, H), f32),
            jax.ShapeDtypeStruct((NP, H), f32),
        ),
        scratch_types=[
            pltpu.VMEM((NCH4, CHUNK), jnp.int32),
            pltpu.VMEM((CHUNK, H), f32),
            pltpu.VMEM_SHARED((NP, H), f32),
            pltpu.SemaphoreType.DMA,
        ],
    )
    def scatter_k(ef_hbm, tr_hbm, row_hbm, aggn_hbm, aggc_hbm,
                  idx, buf, acc, lsem):
        cid = lax.axis_index("c")
        sid = lax.axis_index("s")

        zero16 = jnp.zeros((16,), f32)

        def zrow(r, carry):
            for g in range(H // 16):
                buf[r, pl.ds(g * 16, 16)] = zero16
            return carry

        lax.fori_loop(0, CHUNK, zrow, 0)
        for k in range(NZT // CHUNK):
            pltpu.sync_copy(buf, acc.at[pl.ds(sid * NZT + k * CHUNK, CHUNK)])
        plsc.subcore_barrier()

        pltpu.sync_copy(row_hbm.at[sid], idx)

        def make_loop(src_hbm):
            def chunk(j, carry):
                off = sid * EPT + j * CHUNK
                pltpu.async_copy(src_hbm.at[pl.ds(off, CHUNK)], buf, lsem).wait()
                pltpu.sync_copy(buf, acc.at[idx.at[j]], add=True)
                return carry
            return chunk

        @pl.when(cid == 0)
        def _():
            lax.fori_loop(0, NCH4, make_loop(ef_hbm), 0)

        @pl.when(cid == 1)
        def _():
            lax.fori_loop(0, NCH4, make_loop(tr_hbm), 0)

        plsc.subcore_barrier()
        sl = pl.ds(sid * NZT, NZT)

        @pl.when(cid == 0)
        def _():
            pltpu.sync_copy(acc.at[sl], aggn_hbm.at[sl])

        @pl.when(cid == 1)
        def _():
            pltpu.sync_copy(acc.at[sl], aggc_hbm.at[sl])

    AGGN, AGGC = scatter_k(EF, TR, row16)

    def node_body(h_ref, co_ref, an_ref, ac_ref,
                  wa_ref, wb_ref, b1_ref, g1_ref, bb1_ref,
                  w2_ref, b2_ref, out_ref, cout_ref):
        hh = h_ref[...]
        x = (jnp.dot(hh, wa_ref[...], preferred_element_type=f32)
             + jnp.dot(an_ref[...], wb_ref[...], preferred_element_type=f32)
             + b1_ref[...])
        nn = _silu(_ln_rows(x, g1_ref[...], bb1_ref[...]))
        out_ref[...] = hh + jnp.dot(nn, w2_ref[...], preferred_element_type=f32) + b2_ref[...]
        cout_ref[...] = co_ref[...] + ac_ref[:, 0:3]

    out, coord_out = pl.pallas_call(
        node_body,
        grid=(N // NBLK5,),
        in_specs=[
            pl.BlockSpec((NBLK5, H), lambda i: (i, 0)),
            pl.BlockSpec((NBLK5, 3), lambda i: (i, 0)),
            pl.BlockSpec((NBLK5, H), lambda i: (i, 0)),
            pl.BlockSpec((NBLK5, H), lambda i: (i, 0)),
            pl.BlockSpec((H, H), lambda i: (0, 0)),
            pl.BlockSpec((H, H), lambda i: (0, 0)),
            pl.BlockSpec((1, H), lambda i: (0, 0)),
            pl.BlockSpec((1, H), lambda i: (0, 0)),
            pl.BlockSpec((1, H), lambda i: (0, 0)),
            pl.BlockSpec((H, H), lambda i: (0, 0)),
            pl.BlockSpec((1, H), lambda i: (0, 0)),
        ],
        out_specs=[
            pl.BlockSpec((NBLK5, H), lambda i: (i, 0)),
            pl.BlockSpec((NBLK5, 3), lambda i: (i, 0)),
        ],
        out_shape=[
            jax.ShapeDtypeStruct((N, H), f32),
            jax.ShapeDtypeStruct((N, 3), f32),
        ],
    )(h, coord, AGGN, AGGC,
      wn1a, wn1b, row1(b_n1), row1(g_n1), row1(be_n1),
      W_n2, row1(b_n2))

    return (out, coord_out)


# speedup vs baseline: 3.5085x; 1.7982x over previous
import functools

import jax
import jax.numpy as jnp
from jax import lax
from jax.experimental import pallas as pl
from jax.experimental.pallas import tpu as pltpu
from jax.experimental.pallas import tpu_sc as plsc

NC = 2
NS = 16
NW = NC * NS
CHUNK = 128


def _pick_div(n, cap, mult=1):
    for d in range(min(n, cap), 0, -1):
        if n % d == 0 and d % mult == 0:
            return d
    return 1


def _ln_rows(x, g, b):
    m = jnp.mean(x, axis=-1, keepdims=True)
    v = jnp.mean((x - m) ** 2, axis=-1, keepdims=True)
    return (x - m) / jnp.sqrt(v + 1e-5) * g + b


def _silu(x):
    return x * jax.nn.sigmoid(x)


def kernel(h, edge_index, coord, edge_attr, W_e1, b_e1, g_e1, be_e1, W_e2,
           b_e2, g_e2, be_e2, W_n1, b_n1, g_n1, be_n1, W_n2, b_n2, W_c1,
           b_c1, g_c1, be_c1, W_c2):
    N, H = h.shape
    E = edge_index.shape[1]
    f32 = jnp.float32

    NCH = -(-E // (NW * CHUNK))
    NCH += NCH % 2
    EP = NW * NCH * CHUNK
    EW = EP // NW
    EPT = EP // NS
    NCH4 = EPT // CHUNK
    NP = -(-(N + 1) // 1024) * 1024
    NZT = NP // NS
    ZB = _pick_div(NZT, 128)
    NBLK = _pick_div(NP, 1024, 8)
    NBLK5 = _pick_div(N, 1024, 8)
    EBLK = _pick_div(EP, 2048, 8)

    row1 = lambda a: a.reshape(1, H)

    hp = jnp.pad(h, ((0, NP - N), (0, 0)))
    coordp = jnp.pad(coord, ((0, NP - N), (0, H - coord.shape[1])))
    pad_idx = N + jnp.arange(EP - E, dtype=edge_index.dtype) % (NP - N)
    eip = jnp.concatenate(
        [edge_index, jnp.broadcast_to(pad_idx, (2, EP - E))], axis=1)
    ei32 = eip.reshape(2, NW, NCH, CHUNK)
    row16 = eip[0].reshape(NS, NCH4, CHUNK)
    eap = jnp.pad(edge_attr, ((0, EP - E), (0, 0)))
    w1a = W_e1[0:H]
    w1b = W_e1[H:2 * H]
    w1c = W_e1[2 * H:2 * H + 1]
    w1d = W_e1[2 * H + 1:]
    ED = w1d.shape[0]
    wn1a = W_n1[0:H]
    wn1b = W_n1[H:2 * H]
    wc2r = W_c2.reshape(1, H)

    def pq_body(h_ref, wa_ref, wb_ref, b1_ref, p_ref, q_ref):
        hh = h_ref[...]
        p_ref[...] = jnp.dot(hh, wa_ref[...], preferred_element_type=f32) + b1_ref[...]
        q_ref[...] = jnp.dot(hh, wb_ref[...], preferred_element_type=f32)

    P, Q = pl.pallas_call(
        pq_body,
        grid=(NP // NBLK,),
        in_specs=[
            pl.BlockSpec((NBLK, H), lambda i: (i, 0)),
            pl.BlockSpec((H, H), lambda i: (0, 0)),
            pl.BlockSpec((H, H), lambda i: (0, 0)),
            pl.BlockSpec((1, H), lambda i: (0, 0)),
        ],
        out_specs=[
            pl.BlockSpec((NBLK, H), lambda i: (i, 0)),
            pl.BlockSpec((NBLK, H), lambda i: (i, 0)),
        ],
        out_shape=[
            jax.ShapeDtypeStruct((NP, H), f32),
            jax.ShapeDtypeStruct((NP, H), f32),
        ],
    )(hp, w1a, w1b, row1(b_e1))

    mesh = plsc.VectorSubcoreMesh(core_axis_name="c", subcore_axis_name="s",
                                  num_cores=NC, num_subcores=NS)

    @functools.partial(
        pl.kernel, mesh=mesh,
        out_type=(
            jax.ShapeDtypeStruct((EP, H), f32),
            jax.ShapeDtypeStruct((EP, H), f32),
            jax.ShapeDtypeStruct((EP, H), f32),
            jax.ShapeDtypeStruct((EP, H), f32),
        ),
        scratch_types=[
            pltpu.VMEM((NCH, CHUNK), jnp.int32),
            pltpu.VMEM((NCH, CHUNK), jnp.int32),
            pltpu.VMEM((CHUNK, H), f32),
            pltpu.VMEM((CHUNK, H), f32),
            pltpu.VMEM((CHUNK, H), f32),
            pltpu.VMEM((CHUNK, H), f32),
            pltpu.SemaphoreType.DMA,
        ],
    )
    def gather_k(p_hbm, q_hbm, cp_hbm, ei_hbm, xr_hbm, xc_hbm, cr_hbm, cc_hbm,
                 idxr, idxc, bufp, bufq, bufa, bufb, gsem):
        cid = lax.axis_index("c")
        sid = lax.axis_index("s")
        wid = sid * NC + cid
        base = wid * EW
        pltpu.sync_copy(ei_hbm.at[0, wid], idxr)
        pltpu.sync_copy(ei_hbm.at[1, wid], idxc)

        def chunk(j, carry):
            er = idxr.at[j]
            ec = idxc.at[j]
            c1 = pltpu.async_copy(p_hbm.at[er], bufp, gsem)
            c2 = pltpu.async_copy(q_hbm.at[ec], bufq, gsem)
            c3 = pltpu.async_copy(cp_hbm.at[er], bufa, gsem)
            c4 = pltpu.async_copy(cp_hbm.at[ec], bufb, gsem)
            c1.wait(); c2.wait(); c3.wait(); c4.wait()
            off = base + j * CHUNK
            pltpu.sync_copy(bufp, xr_hbm.at[pl.ds(off, CHUNK)])
            pltpu.sync_copy(bufq, xc_hbm.at[pl.ds(off, CHUNK)])
            pltpu.sync_copy(bufa, cr_hbm.at[pl.ds(off, CHUNK)])
            pltpu.sync_copy(bufb, cc_hbm.at[pl.ds(off, CHUNK)])
            return carry

        lax.fori_loop(0, NCH, chunk, 0)

    XR, XC, CR, CC = gather_k(P, Q, coordp, ei32)

    def edge_body(xr_ref, xc_ref, cr_ref, cc_ref, ea_ref,
                  w1c_ref, w1d_ref, g1_ref, bb1_ref,
                  we2_ref, b2_ref, g2_ref, bb2_ref,
                  wc1_ref, bc1_ref, gc_ref, bbc_ref, wc2_ref,
                  ef_ref, tr_ref):
        cd = cr_ref[...] - cc_ref[...]
        radial = jnp.sum(cd * cd, axis=1, keepdims=True)
        x1 = (xr_ref[...] + xc_ref[...] + radial * w1c_ref[...]
              + jnp.dot(ea_ref[...], w1d_ref[...], preferred_element_type=f32))
        h1 = _silu(_ln_rows(x1, g1_ref[...], bb1_ref[...]))
        x2 = jnp.dot(h1, we2_ref[...], preferred_element_type=f32) + b2_ref[...]
        ef = _silu(_ln_rows(x2, g2_ref[...], bb2_ref[...]))
        ef_ref[...] = ef
        x3 = jnp.dot(ef, wc1_ref[...], preferred_element_type=f32) + bc1_ref[...]
        c1 = _silu(_ln_rows(x3, gc_ref[...], bbc_ref[...]))
        s = jnp.sum(c1 * wc2_ref[...], axis=1, keepdims=True)
        norm = jnp.sqrt(radial + 1e-08)
        tr_ref[...] = cd / (norm + 1.0) * s

    big = lambda: pl.BlockSpec((EBLK, H), lambda i: (i, 0))
    wfull = lambda: pl.BlockSpec((H, H), lambda i: (0, 0))
    prow = lambda: pl.BlockSpec((1, H), lambda i: (0, 0))

    EF, TR = pl.pallas_call(
        edge_body,
        grid=(EP // EBLK,),
        in_specs=[
            big(), big(), big(), big(),
            pl.BlockSpec((EBLK, ED), lambda i: (i, 0)),
            prow(), pl.BlockSpec((ED, H), lambda i: (0, 0)), prow(), prow(),
            wfull(), prow(), prow(), prow(),
            wfull(), prow(), prow(), prow(), prow(),
        ],
        out_specs=[big(), big()],
        out_shape=[
            jax.ShapeDtypeStruct((EP, H), f32),
            jax.ShapeDtypeStruct((EP, H), f32),
        ],
    )(XR, XC, CR, CC, eap,
      w1c, w1d, row1(g_e1), row1(be_e1),
      W_e2, row1(b_e2), row1(g_e2), row1(be_e2),
      W_c1, row1(b_c1), row1(g_c1), row1(be_c1), wc2r)

    @functools.partial(
        pl.kernel, mesh=mesh,
        out_type=(
            jax.ShapeDtypeStruct((NP, H), f32),
            jax.ShapeDtypeStruct((NP, H), f32),
        ),
        scratch_types=[
            pltpu.VMEM((NCH4, CHUNK), jnp.int32),
            pltpu.VMEM((CHUNK, H), f32),
            pltpu.VMEM_SHARED((NP, H), f32),
            pltpu.SemaphoreType.DMA,
        ],
    )
    def scatter_k(ef_hbm, tr_hbm, row_hbm, aggn_hbm, aggc_hbm,
                  idx, buf, acc, lsem):
        cid = lax.axis_index("c")
        sid = lax.axis_index("s")

        zero16 = jnp.zeros((16,), f32)

        def zrow(r, carry):
            for g in range(H // 16):
                buf[r, pl.ds(g * 16, 16)] = zero16
            return carry

        lax.fori_loop(0, CHUNK, zrow, 0)
        for k in range(NZT // CHUNK):
            pltpu.sync_copy(buf, acc.at[pl.ds(sid * NZT + k * CHUNK, CHUNK)])
        plsc.subcore_barrier()

        pltpu.sync_copy(row_hbm.at[sid], idx)

        def make_loop(src_hbm):
            def chunk(j, carry):
                off = sid * EPT + j * CHUNK
                pltpu.async_copy(src_hbm.at[pl.ds(off, CHUNK)], buf, lsem).wait()
                pltpu.sync_copy(buf, acc.at[idx.at[j]], add=True)
                return carry
            return chunk

        @pl.when(cid == 0)
        def _():
            lax.fori_loop(0, NCH4, make_loop(ef_hbm), 0)

        @pl.when(cid == 1)
        def _():
            lax.fori_loop(0, NCH4, make_loop(tr_hbm), 0)

        plsc.subcore_barrier()
        sl = pl.ds(sid * NZT, NZT)

        @pl.when(cid == 0)
        def _():
            pltpu.sync_copy(acc.at[sl], aggn_hbm.at[sl])

        @pl.when(cid == 1)
        def _():
            pltpu.sync_copy(acc.at[sl], aggc_hbm.at[sl])

    AGGN, AGGC = scatter_k(EF, TR, row16)

    def node_body(h_ref, co_ref, an_ref, ac_ref,
                  wa_ref, wb_ref, b1_ref, g1_ref, bb1_ref,
                  w2_ref, b2_ref, out_ref, cout_ref):
        hh = h_ref[...]
        x = (jnp.dot(hh, wa_ref[...], preferred_element_type=f32)
             + jnp.dot(an_ref[...], wb_ref[...], preferred_element_type=f32)
             + b1_ref[...])
        nn = _silu(_ln_rows(x, g1_ref[...], bb1_ref[...]))
        out_ref[...] = hh + jnp.dot(nn, w2_ref[...], preferred_element_type=f32) + b2_ref[...]
        cout_ref[...] = co_ref[...] + ac_ref[:, 0:3]

    out, coord_out = pl.pallas_call(
        node_body,
        grid=(N // NBLK5,),
        in_specs=[
            pl.BlockSpec((NBLK5, H), lambda i: (i, 0)),
            pl.BlockSpec((NBLK5, 3), lambda i: (i, 0)),
            pl.BlockSpec((NBLK5, H), lambda i: (i, 0)),
            pl.BlockSpec((NBLK5, H), lambda i: (i, 0)),
            pl.BlockSpec((H, H), lambda i: (0, 0)),
            pl.BlockSpec((H, H), lambda i: (0, 0)),
            pl.BlockSpec((1, H), lambda i: (0, 0)),
            pl.BlockSpec((1, H), lambda i: (0, 0)),
            pl.BlockSpec((1, H), lambda i: (0, 0)),
            pl.BlockSpec((H, H), lambda i: (0, 0)),
            pl.BlockSpec((1, H), lambda i: (0, 0)),
        ],
        out_specs=[
            pl.BlockSpec((NBLK5, H), lambda i: (i, 0)),
            pl.BlockSpec((NBLK5, 3), lambda i: (i, 0)),
        ],
        out_shape=[
            jax.ShapeDtypeStruct((N, H), f32),
            jax.ShapeDtypeStruct((N, 3), f32),
        ],
    )(h, coord, AGGN, AGGC,
      wn1a, wn1b, row1(b_n1), row1(g_n1), row1(be_n1),
      W_n2, row1(b_n2))

    return (out, coord_out)


# speedup vs baseline: 4.2143x; 1.2012x over previous
import functools

import jax
import jax.numpy as jnp
from jax import lax
from jax.experimental import pallas as pl
from jax.experimental.pallas import tpu as pltpu
from jax.experimental.pallas import tpu_sc as plsc

NC = 2
NS = 16
NW = NC * NS
CHUNK = 128
CP = 16


def _pick_div(n, cap, mult=1):
    for d in range(min(n, cap), 0, -1):
        if n % d == 0 and d % mult == 0:
            return d
    return 1


def _ln_rows(x, g, b):
    hh = x.shape[-1]
    ones_col = jnp.ones((hh, 1), jnp.float32)
    m = jnp.dot(x, ones_col, preferred_element_type=jnp.float32) * (1.0 / hh)
    s2 = jnp.dot(x * x, ones_col, preferred_element_type=jnp.float32) * (1.0 / hh)
    v = s2 - m * m
    rstd = lax.rsqrt(v + 1e-5)
    return (x - m) * rstd * g + b


def _silu(x):
    return 0.5 * x * (1.0 + jnp.tanh(0.5 * x))


def kernel(h, edge_index, coord, edge_attr, W_e1, b_e1, g_e1, be_e1, W_e2,
           b_e2, g_e2, be_e2, W_n1, b_n1, g_n1, be_n1, W_n2, b_n2, W_c1,
           b_c1, g_c1, be_c1, W_c2):
    N, H = h.shape
    E = edge_index.shape[1]
    f32 = jnp.float32
    bf16 = jnp.bfloat16

    NCH = -(-E // (NW * CHUNK))
    NCH += NCH % 2
    EP = NW * NCH * CHUNK
    EW = EP // NW
    EH = EP // NC
    NCH4 = EH // NS // CHUNK
    NP = -(-(N + 1) // 1024) * 1024
    NZT = NP // NS
    NBLK = _pick_div(NP, 1024, 8)
    NBLK5 = _pick_div(N, 1024, 8)
    EBLK = _pick_div(EP, 2048, 8)
    EPB = EP // EBLK

    row1 = lambda a: a.reshape(1, H)

    hp = jnp.pad(h, ((0, NP - N), (0, 0)))
    coordp = jnp.pad(coord, ((0, NP - N), (0, CP - coord.shape[1])))
    pad_idx = N + jnp.arange(EP - E, dtype=edge_index.dtype) % (NP - N)
    eip = jnp.concatenate(
        [edge_index, jnp.broadcast_to(pad_idx, (2, EP - E))], axis=1)
    ei32 = eip.reshape(2, NW, NCH, CHUNK)
    row4 = eip[0].reshape(NC, NS, NCH4, CHUNK)
    eap = jnp.pad(edge_attr, ((0, EP - E), (0, 0)))
    w1a = W_e1[0:H]
    w1b = W_e1[H:2 * H]
    w1c = W_e1[2 * H:2 * H + 1]
    w1d = W_e1[2 * H + 1:]
    ED = w1d.shape[0]
    wn1a = W_n1[0:H]
    wn1b = W_n1[H:2 * H]
    wc2r = W_c2.reshape(1, H)

    def pq_body(h_ref, wa_ref, wb_ref, b1_ref, p_ref, q_ref):
        hh = h_ref[...]
        p_ref[...] = jnp.dot(hh, wa_ref[...], preferred_element_type=f32) + b1_ref[...]
        q_ref[...] = jnp.dot(hh, wb_ref[...], preferred_element_type=f32)

    P, Q = pl.pallas_call(
        pq_body,
        grid=(NP // NBLK,),
        in_specs=[
            pl.BlockSpec((NBLK, H), lambda i: (i, 0)),
            pl.BlockSpec((H, H), lambda i: (0, 0)),
            pl.BlockSpec((H, H), lambda i: (0, 0)),
            pl.BlockSpec((1, H), lambda i: (0, 0)),
        ],
        out_specs=[
            pl.BlockSpec((NBLK, H), lambda i: (i, 0)),
            pl.BlockSpec((NBLK, H), lambda i: (i, 0)),
        ],
        out_shape=[
            jax.ShapeDtypeStruct((NP, H), f32),
            jax.ShapeDtypeStruct((NP, H), f32),
        ],
    )(hp, w1a, w1b, row1(b_e1))

    mesh = plsc.VectorSubcoreMesh(core_axis_name="c", subcore_axis_name="s",
                                  num_cores=NC, num_subcores=NS)

    @functools.partial(
        pl.kernel, mesh=mesh,
        out_type=(
            jax.ShapeDtypeStruct((EP, H), f32),
            jax.ShapeDtypeStruct((EP, H), f32),
        ),
        scratch_types=[
            pltpu.VMEM((NCH, CHUNK), jnp.int32),
            pltpu.VMEM((NCH, CHUNK), jnp.int32),
            pltpu.VMEM((2, CHUNK, H), f32),
            pltpu.VMEM((2, CHUNK, H), f32),
            pltpu.SemaphoreType.DMA,
            pltpu.SemaphoreType.DMA,
            pltpu.SemaphoreType.DMA,
            pltpu.SemaphoreType.DMA,
        ],
    )
    def gather_pq(p_hbm, q_hbm, ei_hbm, xr_hbm, xc_hbm,
                  idxr, idxc, bufp, bufq, gs0, gs1, ws0, ws1):
        cid = lax.axis_index("c")
        sid = lax.axis_index("s")
        wid = sid * NC + cid
        base = wid * EW
        pltpu.sync_copy(ei_hbm.at[0, wid], idxr)
        pltpu.sync_copy(ei_hbm.at[1, wid], idxc)
        gs = (gs0, gs1)
        ws = (ws0, ws1)

        def fire_gather(j, b):
            pltpu.async_copy(p_hbm.at[idxr.at[j]], bufp.at[b], gs[b])
            pltpu.async_copy(q_hbm.at[idxc.at[j]], bufq.at[b], gs[b])

        def wait_gather(b):
            pltpu.make_async_copy(p_hbm.at[idxr.at[0]], bufp.at[b], gs[b]).wait()
            pltpu.make_async_copy(q_hbm.at[idxc.at[0]], bufq.at[b], gs[b]).wait()

        def fire_write(j, b):
            off = base + j * CHUNK
            pltpu.async_copy(bufp.at[b], xr_hbm.at[pl.ds(off, CHUNK)], ws[b])
            pltpu.async_copy(bufq.at[b], xc_hbm.at[pl.ds(off, CHUNK)], ws[b])

        def wait_write(b):
            pltpu.make_async_copy(bufp.at[b], xr_hbm.at[pl.ds(0, CHUNK)], ws[b]).wait()
            pltpu.make_async_copy(bufq.at[b], xc_hbm.at[pl.ds(0, CHUNK)], ws[b]).wait()

        fire_gather(0, 0)

        def step(j, b):
            wait_gather(b)
            fire_write(j, b)

            @pl.when(j + 1 < NCH)
            def _():
                @pl.when(j >= 1)
                def _():
                    wait_write(1 - b)
                fire_gather(j + 1, 1 - b)

        def body(j, carry):
            @pl.when(j % 2 == 0)
            def _():
                step(j, 0)

            @pl.when(j % 2 == 1)
            def _():
                step(j, 1)

            return carry

        lax.fori_loop(0, NCH, body, 0)
        wait_write(0)
        wait_write(1)

    XR, XC = gather_pq(P, Q, ei32)

    @functools.partial(
        pl.kernel, mesh=mesh,
        out_type=(
            jax.ShapeDtypeStruct((EP, CP), f32),
            jax.ShapeDtypeStruct((EPB, 1, EBLK), f32),
        ),
        scratch_types=[
            pltpu.VMEM((NCH, CHUNK), jnp.int32),
            pltpu.VMEM((NCH, CHUNK), jnp.int32),
            pltpu.VMEM((CHUNK, CP), f32),
            pltpu.VMEM((CHUNK, CP), f32),
            pltpu.VMEM((1, CHUNK), f32),
            pltpu.SemaphoreType.DMA,
        ],
        compiler_params=pltpu.CompilerParams(use_tc_tiling_on_sc=False,
                                             needs_layout_passes=False),
    )
    def gather_cd(cp_hbm, ei_hbm, cd_hbm, rad_hbm,
                  idxr, idxc, bufa, bufb, radb, sem):
        cid = lax.axis_index("c")
        sid = lax.axis_index("s")
        wid = sid * NC + cid
        base = wid * EW
        pltpu.sync_copy(ei_hbm.at[0, wid], idxr)
        pltpu.sync_copy(ei_hbm.at[1, wid], idxc)
        lanes = lax.iota(jnp.int32, 16)

        def chunk(j, carry):
            c1 = pltpu.async_copy(cp_hbm.at[idxr.at[j]], bufa, sem)
            c2 = pltpu.async_copy(cp_hbm.at[idxc.at[j]], bufb, sem)
            c1.wait(); c2.wait()

            def group(k, carry2):
                acc = jnp.zeros((16,), f32)
                for u in range(16):
                    r = k * 16 + u
                    v = bufa[r, :] - bufb[r, :]
                    bufa[r, :] = v
                    s = jnp.sum(v * v)
                    acc = jnp.where(lanes == u, s, acc)
                radb[0, pl.ds(k * 16, 16)] = acc
                return carry2

            lax.fori_loop(0, CHUNK // 16, group, 0)
            off = base + j * CHUNK
            pltpu.sync_copy(bufa, cd_hbm.at[pl.ds(off, CHUNK)])
            pltpu.sync_copy(
                radb,
                rad_hbm.at[off // EBLK, pl.ds(0, 1), pl.ds(off % EBLK, CHUNK)])
            return carry

        lax.fori_loop(0, NCH, chunk, 0)

    CD, RAD = gather_cd(coordp, ei32)

    def edge_body(xr_ref, xc_ref, rad_ref, ea_ref,
                  w1c_ref, w1d_ref, g1_ref, bb1_ref,
                  we2_ref, b2_ref, g2_ref, bb2_ref,
                  wc1_ref, bc1_ref, gc_ref, bbc_ref, wc2_ref,
                  ef_ref, ts_ref):
        radial = jnp.swapaxes(rad_ref[0], 0, 1)
        x1 = (xr_ref[...] + xc_ref[...] + radial * w1c_ref[...]
              + jnp.dot(ea_ref[...], w1d_ref[...], preferred_element_type=f32))
        h1 = _silu(_ln_rows(x1, g1_ref[...], bb1_ref[...]))
        x2 = jnp.dot(h1.astype(bf16), we2_ref[...].astype(bf16),
                     preferred_element_type=f32) + b2_ref[...]
        ef = _silu(_ln_rows(x2, g2_ref[...], bb2_ref[...]))
        ef_ref[...] = ef
        x3 = jnp.dot(ef.astype(bf16), wc1_ref[...].astype(bf16),
                     preferred_element_type=f32) + bc1_ref[...]
        c1 = _silu(_ln_rows(x3, gc_ref[...], bbc_ref[...]))
        s = jnp.dot(c1, wc2_ref[...], preferred_element_type=f32)
        t = s / (jnp.sqrt(radial + 1e-08) + 1.0)
        ts_ref[...] = jnp.swapaxes(t, 0, 1).reshape(1, 1, EBLK)

    big = lambda: pl.BlockSpec((EBLK, H), lambda i: (i, 0))
    wfull = lambda: pl.BlockSpec((H, H), lambda i: (0, 0))
    prow = lambda: pl.BlockSpec((1, H), lambda i: (0, 0))

    EF, TS = pl.pallas_call(
        edge_body,
        grid=(EP // EBLK,),
        in_specs=[
            big(), big(),
            pl.BlockSpec((1, 1, EBLK), lambda i: (i, 0, 0)),
            pl.BlockSpec((EBLK, ED), lambda i: (i, 0)),
            prow(), pl.BlockSpec((ED, H), lambda i: (0, 0)), prow(), prow(),
            wfull(), prow(), prow(), prow(),
            wfull(), prow(), prow(), prow(),
            pl.BlockSpec((H, 1), lambda i: (0, 0)),
        ],
        out_specs=[big(), pl.BlockSpec((1, 1, EBLK), lambda i: (i, 0, 0))],
        out_shape=[
            jax.ShapeDtypeStruct((EP, H), f32),
            jax.ShapeDtypeStruct((EPB, 1, EBLK), f32),
        ],
    )(XR, XC, RAD, eap,
      w1c, w1d, row1(g_e1), row1(be_e1),
      W_e2, row1(b_e2), row1(g_e2), row1(be_e2),
      W_c1, row1(b_c1), row1(g_c1), row1(be_c1), W_c2)

    @functools.partial(
        pl.kernel, mesh=mesh,
        out_type=jax.ShapeDtypeStruct((NC, NP, H), f32),
        scratch_types=[
            pltpu.VMEM((NCH4, CHUNK), jnp.int32),
            pltpu.VMEM((2, CHUNK, H), f32),
            pltpu.VMEM_SHARED((NP, H), f32),
            pltpu.SemaphoreType.DMA,
            pltpu.SemaphoreType.DMA,
        ],
    )
    def scatter_n(ef_hbm, row_hbm, aggn_hbm, idx, buf, acc, ls0, ls1):
        cid = lax.axis_index("c")
        sid = lax.axis_index("s")
        zero16 = jnp.zeros((16,), f32)
        ls = (ls0, ls1)

        def zrow(r, carry):
            for g in range(H // 16):
                buf[0, r, pl.ds(g * 16, 16)] = zero16
            return carry

        lax.fori_loop(0, CHUNK, zrow, 0)
        for k in range(NZT // CHUNK):
            pltpu.sync_copy(buf.at[0], acc.at[pl.ds(sid * NZT + k * CHUNK, CHUNK)])
        plsc.subcore_barrier()

        pltpu.sync_copy(row_hbm.at[cid, sid], idx)
        base = cid * EH + sid * (EH // NS)

        def fire_load(j, b):
            off = base + j * CHUNK
            pltpu.async_copy(ef_hbm.at[pl.ds(off, CHUNK)], buf.at[b], ls[b])

        def wait_load(b):
            pltpu.make_async_copy(
                ef_hbm.at[pl.ds(0, CHUNK)], buf.at[b], ls[b]).wait()

        fire_load(0, 0)

        def step(j, b):
            wait_load(b)

            @pl.when(j + 1 < NCH4)
            def _():
                fire_load(j + 1, 1 - b)

            pltpu.sync_copy(buf.at[b], acc.at[idx.at[j]], add=True)

        def chunk(j, carry):
            @pl.when(j % 2 == 0)
            def _():
                step(j, 0)

            @pl.when(j % 2 == 1)
            def _():
                step(j, 1)

            return carry

        lax.fori_loop(0, NCH4, chunk, 0)
        plsc.subcore_barrier()
        sl = pl.ds(sid * NZT, NZT)
        pltpu.sync_copy(acc.at[sl], aggn_hbm.at[cid, sl])

    AGGN = scatter_n(EF, row4)

    @functools.partial(
        pl.kernel, mesh=mesh,
        out_type=jax.ShapeDtypeStruct((NC, NP, CP), f32),
        scratch_types=[
            pltpu.VMEM((NCH4, CHUNK), jnp.int32),
            pltpu.VMEM((CHUNK, CP), f32),
            pltpu.VMEM((1, CHUNK), f32),
            pltpu.VMEM_SHARED((NP, CP), f32),
            pltpu.SemaphoreType.DMA,
        ],
        compiler_params=pltpu.CompilerParams(use_tc_tiling_on_sc=False,
                                             needs_layout_passes=False),
    )
    def scatter_c(cd_hbm, ts_hbm, row_hbm, aggc_hbm, idx, buf, tsb, acc, lsem):
        cid = lax.axis_index("c")
        sid = lax.axis_index("s")
        zero16 = jnp.zeros((16,), f32)

        def zrow(r, carry):
            buf[r, :] = zero16
            return carry

        lax.fori_loop(0, CHUNK, zrow, 0)
        for k in range(NZT // CHUNK):
            pltpu.sync_copy(buf, acc.at[pl.ds(sid * NZT + k * CHUNK, CHUNK)])
        plsc.subcore_barrier()

        pltpu.sync_copy(row_hbm.at[cid, sid], idx)
        base = cid * EH + sid * (EH // NS)
        lanes = lax.iota(jnp.int32, 16)

        def chunk(j, carry):
            off = base + j * CHUNK
            c1 = pltpu.async_copy(cd_hbm.at[pl.ds(off, CHUNK)], buf, lsem)
            c2 = pltpu.async_copy(
                ts_hbm.at[off // EBLK, pl.ds(0, 1), pl.ds(off % EBLK, CHUNK)],
                tsb, lsem)
            c1.wait(); c2.wait()

            def mul(k, carry2):
                tv = tsb[0, pl.ds(k * 16, 16)]
                for u in range(16):
                    r = k * 16 + u
                    t = jnp.sum(jnp.where(lanes == u, tv, 0.0))
                    buf[r, :] = buf[r, :] * t
                return carry2

            lax.fori_loop(0, CHUNK // 16, mul, 0)
            pltpu.sync_copy(buf, acc.at[idx.at[j]], add=True)
            return carry

        lax.fori_loop(0, NCH4, chunk, 0)
        plsc.subcore_barrier()
        sl = pl.ds(sid * NZT, NZT)
        pltpu.sync_copy(acc.at[sl], aggc_hbm.at[cid, sl])

    AGGC = scatter_c(CD, TS, row4)

    def node_body(h_ref, co_ref, an_ref, ac_ref,
                  wa_ref, wb_ref, b1_ref, g1_ref, bb1_ref,
                  w2_ref, b2_ref, out_ref, cout_ref):
        hh = h_ref[...]
        agg = an_ref[0] + an_ref[1]
        x = (jnp.dot(hh, wa_ref[...], preferred_element_type=f32)
             + jnp.dot(agg, wb_ref[...], preferred_element_type=f32)
             + b1_ref[...])
        nn = _silu(_ln_rows(x, g1_ref[...], bb1_ref[...]))
        out_ref[...] = hh + jnp.dot(nn, w2_ref[...], preferred_element_type=f32) + b2_ref[...]
        cout_ref[...] = co_ref[...] + ac_ref[0][:, 0:3] + ac_ref[1][:, 0:3]

    out, coord_out = pl.pallas_call(
        node_body,
        grid=(N // NBLK5,),
        in_specs=[
            pl.BlockSpec((NBLK5, H), lambda i: (i, 0)),
            pl.BlockSpec((NBLK5, 3), lambda i: (i, 0)),
            pl.BlockSpec((NC, NBLK5, H), lambda i: (0, i, 0)),
            pl.BlockSpec((NC, NBLK5, CP), lambda i: (0, i, 0)),
            pl.BlockSpec((H, H), lambda i: (0, 0)),
            pl.BlockSpec((H, H), lambda i: (0, 0)),
            pl.BlockSpec((1, H), lambda i: (0, 0)),
            pl.BlockSpec((1, H), lambda i: (0, 0)),
            pl.BlockSpec((1, H), lambda i: (0, 0)),
            pl.BlockSpec((H, H), lambda i: (0, 0)),
            pl.BlockSpec((1, H), lambda i: (0, 0)),
        ],
        out_specs=[
            pl.BlockSpec((NBLK5, H), lambda i: (i, 0)),
            pl.BlockSpec((NBLK5, 3), lambda i: (i, 0)),
        ],
        out_shape=[
            jax.ShapeDtypeStruct((N, H), f32),
            jax.ShapeDtypeStruct((N, 3), f32),
        ],
    )(h, coord, AGGN, AGGC,
      wn1a, wn1b, row1(b_n1), row1(g_n1), row1(be_n1),
      W_n2, row1(b_n2))

    return (out, coord_out)


# speedup vs baseline: 4.6141x; 1.0949x over previous
import functools

import jax
import jax.numpy as jnp
from jax import lax
from jax.experimental import pallas as pl
from jax.experimental.pallas import tpu as pltpu
from jax.experimental.pallas import tpu_sc as plsc

NC = 2
NS = 16
NW = NC * NS
CHUNK = 128
CP = 16


def _pick_div(n, cap, mult=1):
    for d in range(min(n, cap), 0, -1):
        if n % d == 0 and d % mult == 0:
            return d
    return 1


def _ln_rows(x, g, b):
    hh = x.shape[-1]
    ones_col = jnp.ones((hh, 1), jnp.float32)
    m = jnp.dot(x, ones_col, preferred_element_type=jnp.float32) * (1.0 / hh)
    s2 = jnp.dot(x * x, ones_col, preferred_element_type=jnp.float32) * (1.0 / hh)
    v = s2 - m * m
    rstd = lax.rsqrt(v + 1e-5)
    return (x - m) * rstd * g + b


def _silu(x):
    return 0.5 * x * (1.0 + jnp.tanh(0.5 * x))


def kernel(h, edge_index, coord, edge_attr, W_e1, b_e1, g_e1, be_e1, W_e2,
           b_e2, g_e2, be_e2, W_n1, b_n1, g_n1, be_n1, W_n2, b_n2, W_c1,
           b_c1, g_c1, be_c1, W_c2):
    N, H = h.shape
    E = edge_index.shape[1]
    f32 = jnp.float32
    bf16 = jnp.bfloat16

    NCH = -(-E // (NW * CHUNK))
    NCH += NCH % 2
    EP = NW * NCH * CHUNK
    EW = EP // NW
    EH = EP // NC
    NCH4 = EH // NS // CHUNK
    NP = -(-(N + 1) // 1024) * 1024
    NZT = NP // NS
    NBLK = _pick_div(NP, 1024, 8)
    NBLK5 = _pick_div(N, 1024, 8)
    EBLK = _pick_div(EP, 2048, 8)
    EPB = EP // EBLK

    row1 = lambda a: a.reshape(1, H)

    hp = jnp.pad(h, ((0, NP - N), (0, 0)))
    coordp = jnp.pad(coord, ((0, NP - N), (0, CP - coord.shape[1])))
    pad_idx = N + jnp.arange(EP - E, dtype=edge_index.dtype) % (NP - N)
    eip = jnp.concatenate(
        [edge_index, jnp.broadcast_to(pad_idx, (2, EP - E))], axis=1)
    ei32 = eip.reshape(2, NW, NCH, CHUNK)
    row4 = eip[0].reshape(NC, NS, NCH4, CHUNK)
    eap = jnp.pad(edge_attr, ((0, EP - E), (0, 0))).astype(bf16)
    w1a = W_e1[0:H]
    w1b = W_e1[H:2 * H]
    w1c = W_e1[2 * H:2 * H + 1]
    w1d = W_e1[2 * H + 1:]
    ED = w1d.shape[0]
    wn1a = W_n1[0:H]
    wn1b = W_n1[H:2 * H]
    wc2r = W_c2.reshape(1, H)

    def pq_body(h_ref, wa_ref, wb_ref, b1_ref, p_ref, q_ref):
        hh = h_ref[...]
        p_ref[...] = jnp.dot(hh, wa_ref[...], preferred_element_type=f32) + b1_ref[...]
        q_ref[...] = jnp.dot(hh, wb_ref[...], preferred_element_type=f32)

    P, Q = pl.pallas_call(
        pq_body,
        grid=(NP // NBLK,),
        in_specs=[
            pl.BlockSpec((NBLK, H), lambda i: (i, 0)),
            pl.BlockSpec((H, H), lambda i: (0, 0)),
            pl.BlockSpec((H, H), lambda i: (0, 0)),
            pl.BlockSpec((1, H), lambda i: (0, 0)),
        ],
        out_specs=[
            pl.BlockSpec((NBLK, H), lambda i: (i, 0)),
            pl.BlockSpec((NBLK, H), lambda i: (i, 0)),
        ],
        out_shape=[
            jax.ShapeDtypeStruct((NP, H), f32),
            jax.ShapeDtypeStruct((NP, H), f32),
        ],
    )(hp, w1a, w1b, row1(b_e1))

    mesh = plsc.VectorSubcoreMesh(core_axis_name="c", subcore_axis_name="s",
                                  num_cores=NC, num_subcores=NS)

    @functools.partial(
        pl.kernel, mesh=mesh,
        out_type=jax.ShapeDtypeStruct((EP, H), f32),
        scratch_types=[
            pltpu.VMEM((NCH, CHUNK), jnp.int32),
            pltpu.VMEM((NCH, CHUNK), jnp.int32),
            pltpu.VMEM((2, CHUNK, H), f32),
            pltpu.VMEM((2, CHUNK, H), f32),
            pltpu.SemaphoreType.DMA,
            pltpu.SemaphoreType.DMA,
            pltpu.SemaphoreType.DMA,
            pltpu.SemaphoreType.DMA,
        ],
    )
    def gather_pq(p_hbm, q_hbm, ei_hbm, x_hbm,
                  idxr, idxc, bufp, bufq, gs0, gs1, ws0, ws1):
        cid = lax.axis_index("c")
        sid = lax.axis_index("s")
        wid = sid * NC + cid
        base = wid * EW
        pltpu.sync_copy(ei_hbm.at[0, wid], idxr)
        pltpu.sync_copy(ei_hbm.at[1, wid], idxc)
        gs = (gs0, gs1)
        ws = (ws0, ws1)

        def fire_gather(j, b):
            pltpu.async_copy(p_hbm.at[idxr.at[j]], bufp.at[b], gs[b])
            pltpu.async_copy(q_hbm.at[idxc.at[j]], bufq.at[b], gs[b])

        def wait_gather(b):
            pltpu.make_async_copy(p_hbm.at[idxr.at[0]], bufp.at[b], gs[b]).wait()
            pltpu.make_async_copy(q_hbm.at[idxc.at[0]], bufq.at[b], gs[b]).wait()

        def fire_write(j, b):
            off = base + j * CHUNK
            pltpu.async_copy(bufp.at[b], x_hbm.at[pl.ds(off, CHUNK)], ws[b])

        def wait_write(b):
            pltpu.make_async_copy(bufp.at[b], x_hbm.at[pl.ds(0, CHUNK)], ws[b]).wait()

        fire_gather(0, 0)

        def step(j, b):
            wait_gather(b)

            @pl.when(j + 1 < NCH)
            def _():
                @pl.when(j >= 1)
                def _():
                    wait_write(1 - b)
                fire_gather(j + 1, 1 - b)

            def addrow(r, carry2):
                for g in range(H // 16):
                    sl = pl.ds(g * 16, 16)
                    bufp[b, r, sl] = bufp[b, r, sl] + bufq[b, r, sl]
                return carry2

            lax.fori_loop(0, CHUNK, addrow, 0)
            fire_write(j, b)

        def body(j, carry):
            @pl.when(j % 2 == 0)
            def _():
                step(j, 0)

            @pl.when(j % 2 == 1)
            def _():
                step(j, 1)

            return carry

        lax.fori_loop(0, NCH, body, 0)
        wait_write(0)
        wait_write(1)

    X = gather_pq(P, Q, ei32)

    @functools.partial(
        pl.kernel, mesh=mesh,
        out_type=(
            jax.ShapeDtypeStruct((EP, CP), f32),
            jax.ShapeDtypeStruct((EPB, 1, EBLK), f32),
        ),
        scratch_types=[
            pltpu.VMEM((NCH, CHUNK), jnp.int32),
            pltpu.VMEM((NCH, CHUNK), jnp.int32),
            pltpu.VMEM((CHUNK, CP), f32),
            pltpu.VMEM((CHUNK, CP), f32),
            pltpu.VMEM((1, CHUNK), f32),
            pltpu.SemaphoreType.DMA,
        ],
        compiler_params=pltpu.CompilerParams(use_tc_tiling_on_sc=False,
                                             needs_layout_passes=False),
    )
    def gather_cd(cp_hbm, ei_hbm, cd_hbm, rad_hbm,
                  idxr, idxc, bufa, bufb, radb, sem):
        cid = lax.axis_index("c")
        sid = lax.axis_index("s")
        wid = sid * NC + cid
        base = wid * EW
        pltpu.sync_copy(ei_hbm.at[0, wid], idxr)
        pltpu.sync_copy(ei_hbm.at[1, wid], idxc)
        lanes = lax.iota(jnp.int32, 16)

        def chunk(j, carry):
            c1 = pltpu.async_copy(cp_hbm.at[idxr.at[j]], bufa, sem)
            c2 = pltpu.async_copy(cp_hbm.at[idxc.at[j]], bufb, sem)
            c1.wait(); c2.wait()

            def group(k, carry2):
                acc = jnp.zeros((16,), f32)
                for u in range(16):
                    r = k * 16 + u
                    v = bufa[r, :] - bufb[r, :]
                    bufa[r, :] = v
                    s = jnp.sum(v * v)
                    acc = jnp.where(lanes == u, s, acc)
                radb[0, pl.ds(k * 16, 16)] = acc
                return carry2

            lax.fori_loop(0, CHUNK // 16, group, 0)
            off = base + j * CHUNK
            pltpu.sync_copy(bufa, cd_hbm.at[pl.ds(off, CHUNK)])
            pltpu.sync_copy(
                radb,
                rad_hbm.at[off // EBLK, pl.ds(0, 1), pl.ds(off % EBLK, CHUNK)])
            return carry

        lax.fori_loop(0, NCH, chunk, 0)

    CD, RAD = gather_cd(coordp, ei32)

    def edge_body(x_ref, rad_ref, ea_ref,
                  w1c_ref, w1d_ref, g1_ref, bb1_ref,
                  we2_ref, b2_ref, g2_ref, bb2_ref,
                  wc1_ref, bc1_ref, gc_ref, bbc_ref, wc2_ref,
                  ef_ref, ts_ref):
        radial = jnp.swapaxes(rad_ref[0], 0, 1)
        x1 = (x_ref[...] + radial * w1c_ref[...]
              + jnp.dot(ea_ref[...], w1d_ref[...].astype(bf16),
                        preferred_element_type=f32))
        h1 = _silu(_ln_rows(x1, g1_ref[...], bb1_ref[...]))
        x2 = jnp.dot(h1.astype(bf16), we2_ref[...].astype(bf16),
                     preferred_element_type=f32) + b2_ref[...]
        ef = _silu(_ln_rows(x2, g2_ref[...], bb2_ref[...]))
        ef_ref[...] = ef
        x3 = jnp.dot(ef.astype(bf16), wc1_ref[...].astype(bf16),
                     preferred_element_type=f32) + bc1_ref[...]
        c1 = _silu(_ln_rows(x3, gc_ref[...], bbc_ref[...]))
        s = jnp.dot(c1, wc2_ref[...], preferred_element_type=f32)
        t = s / (jnp.sqrt(radial + 1e-08) + 1.0)
        ts_ref[...] = jnp.swapaxes(t, 0, 1).reshape(1, 1, EBLK)

    big = lambda: pl.BlockSpec((EBLK, H), lambda i: (i, 0))
    wfull = lambda: pl.BlockSpec((H, H), lambda i: (0, 0))
    prow = lambda: pl.BlockSpec((1, H), lambda i: (0, 0))

    EF, TS = pl.pallas_call(
        edge_body,
        grid=(EP // EBLK,),
        in_specs=[
            big(),
            pl.BlockSpec((1, 1, EBLK), lambda i: (i, 0, 0)),
            pl.BlockSpec((EBLK, ED), lambda i: (i, 0)),
            prow(), pl.BlockSpec((ED, H), lambda i: (0, 0)), prow(), prow(),
            wfull(), prow(), prow(), prow(),
            wfull(), prow(), prow(), prow(),
            pl.BlockSpec((H, 1), lambda i: (0, 0)),
        ],
        out_specs=[big(), pl.BlockSpec((1, 1, EBLK), lambda i: (i, 0, 0))],
        out_shape=[
            jax.ShapeDtypeStruct((EP, H), f32),
            jax.ShapeDtypeStruct((EPB, 1, EBLK), f32),
        ],
    )(X, RAD, eap,
      w1c, w1d, row1(g_e1), row1(be_e1),
      W_e2, row1(b_e2), row1(g_e2), row1(be_e2),
      W_c1, row1(b_c1), row1(g_c1), row1(be_c1), W_c2)

    @functools.partial(
        pl.kernel, mesh=mesh,
        out_type=jax.ShapeDtypeStruct((NC, NP, H), f32),
        scratch_types=[
            pltpu.VMEM((NCH4, CHUNK), jnp.int32),
            pltpu.VMEM((2, CHUNK, H), f32),
            pltpu.VMEM_SHARED((NP, H), f32),
            pltpu.SemaphoreType.DMA,
            pltpu.SemaphoreType.DMA,
        ],
    )
    def scatter_n(ef_hbm, row_hbm, aggn_hbm, idx, buf, acc, ls0, ls1):
        cid = lax.axis_index("c")
        sid = lax.axis_index("s")
        zero16 = jnp.zeros((16,), f32)
        ls = (ls0, ls1)

        def zrow(r, carry):
            for g in range(H // 16):
                buf[0, r, pl.ds(g * 16, 16)] = zero16
            return carry

        lax.fori_loop(0, CHUNK, zrow, 0)
        for k in range(NZT // CHUNK):
            pltpu.sync_copy(buf.at[0], acc.at[pl.ds(sid * NZT + k * CHUNK, CHUNK)])
        plsc.subcore_barrier()

        pltpu.sync_copy(row_hbm.at[cid, sid], idx)
        base = cid * EH + sid * (EH // NS)

        def fire_load(j, b):
            off = base + j * CHUNK
            pltpu.async_copy(ef_hbm.at[pl.ds(off, CHUNK)], buf.at[b], ls[b])

        def wait_load(b):
            pltpu.make_async_copy(
                ef_hbm.at[pl.ds(0, CHUNK)], buf.at[b], ls[b]).wait()

        fire_load(0, 0)

        def step(j, b):
            wait_load(b)

            @pl.when(j + 1 < NCH4)
            def _():
                fire_load(j + 1, 1 - b)

            pltpu.sync_copy(buf.at[b], acc.at[idx.at[j]], add=True)

        def chunk(j, carry):
            @pl.when(j % 2 == 0)
            def _():
                step(j, 0)

            @pl.when(j % 2 == 1)
            def _():
                step(j, 1)

            return carry

        lax.fori_loop(0, NCH4, chunk, 0)
        plsc.subcore_barrier()
        sl = pl.ds(sid * NZT, NZT)
        pltpu.sync_copy(acc.at[sl], aggn_hbm.at[cid, sl])

    AGGN = scatter_n(EF, row4)

    @functools.partial(
        pl.kernel, mesh=mesh,
        out_type=jax.ShapeDtypeStruct((NC, NP, CP), f32),
        scratch_types=[
            pltpu.VMEM((NCH4, CHUNK), jnp.int32),
            pltpu.VMEM((CHUNK, CP), f32),
            pltpu.VMEM((1, CHUNK), f32),
            pltpu.VMEM_SHARED((NP, CP), f32),
            pltpu.SemaphoreType.DMA,
        ],
        compiler_params=pltpu.CompilerParams(use_tc_tiling_on_sc=False,
                                             needs_layout_passes=False),
    )
    def scatter_c(cd_hbm, ts_hbm, row_hbm, aggc_hbm, idx, buf, tsb, acc, lsem):
        cid = lax.axis_index("c")
        sid = lax.axis_index("s")
        zero16 = jnp.zeros((16,), f32)

        def zrow(r, carry):
            buf[r, :] = zero16
            return carry

        lax.fori_loop(0, CHUNK, zrow, 0)
        for k in range(NZT // CHUNK):
            pltpu.sync_copy(buf, acc.at[pl.ds(sid * NZT + k * CHUNK, CHUNK)])
        plsc.subcore_barrier()

        pltpu.sync_copy(row_hbm.at[cid, sid], idx)
        base = cid * EH + sid * (EH // NS)
        lanes = lax.iota(jnp.int32, 16)

        def chunk(j, carry):
            off = base + j * CHUNK
            c1 = pltpu.async_copy(cd_hbm.at[pl.ds(off, CHUNK)], buf, lsem)
            c2 = pltpu.async_copy(
                ts_hbm.at[off // EBLK, pl.ds(0, 1), pl.ds(off % EBLK, CHUNK)],
                tsb, lsem)
            c1.wait(); c2.wait()

            def mul(k, carry2):
                tv = tsb[0, pl.ds(k * 16, 16)]
                for u in range(16):
                    r = k * 16 + u
                    t = jnp.sum(jnp.where(lanes == u, tv, 0.0))
                    buf[r, :] = buf[r, :] * t
                return carry2

            lax.fori_loop(0, CHUNK // 16, mul, 0)
            pltpu.sync_copy(buf, acc.at[idx.at[j]], add=True)
            return carry

        lax.fori_loop(0, NCH4, chunk, 0)
        plsc.subcore_barrier()
        sl = pl.ds(sid * NZT, NZT)
        pltpu.sync_copy(acc.at[sl], aggc_hbm.at[cid, sl])

    AGGC = scatter_c(CD, TS, row4)

    def node_body(h_ref, co_ref, an_ref, ac_ref,
                  wa_ref, wb_ref, b1_ref, g1_ref, bb1_ref,
                  w2_ref, b2_ref, out_ref, cout_ref):
        hh = h_ref[...]
        agg = an_ref[0] + an_ref[1]
        x = (jnp.dot(hh, wa_ref[...], preferred_element_type=f32)
             + jnp.dot(agg, wb_ref[...], preferred_element_type=f32)
             + b1_ref[...])
        nn = _silu(_ln_rows(x, g1_ref[...], bb1_ref[...]))
        out_ref[...] = hh + jnp.dot(nn, w2_ref[...], preferred_element_type=f32) + b2_ref[...]
        cout_ref[...] = co_ref[...] + ac_ref[0][:, 0:3] + ac_ref[1][:, 0:3]

    out, coord_out = pl.pallas_call(
        node_body,
        grid=(N // NBLK5,),
        in_specs=[
            pl.BlockSpec((NBLK5, H), lambda i: (i, 0)),
            pl.BlockSpec((NBLK5, 3), lambda i: (i, 0)),
            pl.BlockSpec((NC, NBLK5, H), lambda i: (0, i, 0)),
            pl.BlockSpec((NC, NBLK5, CP), lambda i: (0, i, 0)),
            pl.BlockSpec((H, H), lambda i: (0, 0)),
            pl.BlockSpec((H, H), lambda i: (0, 0)),
            pl.BlockSpec((1, H), lambda i: (0, 0)),
            pl.BlockSpec((1, H), lambda i: (0, 0)),
            pl.BlockSpec((1, H), lambda i: (0, 0)),
            pl.BlockSpec((H, H), lambda i: (0, 0)),
            pl.BlockSpec((1, H), lambda i: (0, 0)),
        ],
        out_specs=[
            pl.BlockSpec((NBLK5, H), lambda i: (i, 0)),
            pl.BlockSpec((NBLK5, 3), lambda i: (i, 0)),
        ],
        out_shape=[
            jax.ShapeDtypeStruct((N, H), f32),
            jax.ShapeDtypeStruct((N, 3), f32),
        ],
    )(h, coord, AGGN, AGGC,
      wn1a, wn1b, row1(b_n1), row1(g_n1), row1(be_n1),
      W_n2, row1(b_n2))

    return (out, coord_out)


# speedup vs baseline: 4.9559x; 1.0741x over previous
import functools

import jax
import jax.numpy as jnp
from jax import lax
from jax.experimental import pallas as pl
from jax.experimental.pallas import tpu as pltpu
from jax.experimental.pallas import tpu_sc as plsc

NC = 2
NS = 16
NW = NC * NS
CHUNK = 128
CP = 16


def _pick_div(n, cap, mult=1):
    for d in range(min(n, cap), 0, -1):
        if n % d == 0 and d % mult == 0:
            return d
    return 1


def _ln_rows(x, g, b):
    hh = x.shape[-1]
    ones_col = jnp.ones((hh, 1), jnp.float32)
    m = jnp.dot(x, ones_col, preferred_element_type=jnp.float32) * (1.0 / hh)
    s2 = jnp.dot(x * x, ones_col, preferred_element_type=jnp.float32) * (1.0 / hh)
    v = s2 - m * m
    rstd = lax.rsqrt(v + 1e-5)
    return (x - m) * rstd * g + b


def _silu(x):
    return 0.5 * x * (1.0 + jnp.tanh(0.5 * x))


def kernel(h, edge_index, coord, edge_attr, W_e1, b_e1, g_e1, be_e1, W_e2,
           b_e2, g_e2, be_e2, W_n1, b_n1, g_n1, be_n1, W_n2, b_n2, W_c1,
           b_c1, g_c1, be_c1, W_c2):
    N, H = h.shape
    E = edge_index.shape[1]
    f32 = jnp.float32
    bf16 = jnp.bfloat16

    NCH = -(-E // (NW * CHUNK))
    NCH += NCH % 2
    EP = NW * NCH * CHUNK
    EW = EP // NW
    EH = EP // NC
    NCH4 = EH // NS // CHUNK
    NP = -(-(N + 1) // 1024) * 1024
    NZT = NP // NS
    NBLK = _pick_div(NP, 1024, 8)
    NBLK5 = _pick_div(N, 1024, 8)
    EBLK = _pick_div(EP, 2048, 8)
    EPB = EP // EBLK

    row1 = lambda a: a.reshape(1, H)

    hp = jnp.pad(h, ((0, NP - N), (0, 0)))
    coordp = jnp.pad(coord, ((0, NP - N), (0, CP - coord.shape[1])))
    pad_idx = N + jnp.arange(EP - E, dtype=edge_index.dtype) % (NP - N)
    eip = jnp.concatenate(
        [edge_index, jnp.broadcast_to(pad_idx, (2, EP - E))], axis=1)
    ei32 = eip.reshape(2, NW, NCH, CHUNK)
    row4 = eip[0].reshape(NC, NS, NCH4, CHUNK)
    eap = jnp.pad(edge_attr, ((0, EP - E), (0, 0))).astype(bf16)
    w1a = W_e1[0:H]
    w1b = W_e1[H:2 * H]
    w1c = W_e1[2 * H:2 * H + 1]
    w1d = W_e1[2 * H + 1:]
    ED = w1d.shape[0]
    wn1a = W_n1[0:H]
    wn1b = W_n1[H:2 * H]
    wc2r = W_c2.reshape(1, H)

    def pq_body(h_ref, wa_ref, wb_ref, b1_ref, p_ref, q_ref):
        hh = h_ref[...]
        p_ref[...] = jnp.dot(hh, wa_ref[...], preferred_element_type=f32) + b1_ref[...]
        q_ref[...] = jnp.dot(hh, wb_ref[...], preferred_element_type=f32)

    P, Q = pl.pallas_call(
        pq_body,
        grid=(NP // NBLK,),
        in_specs=[
            pl.BlockSpec((NBLK, H), lambda i: (i, 0)),
            pl.BlockSpec((H, H), lambda i: (0, 0)),
            pl.BlockSpec((H, H), lambda i: (0, 0)),
            pl.BlockSpec((1, H), lambda i: (0, 0)),
        ],
        out_specs=[
            pl.BlockSpec((NBLK, H), lambda i: (i, 0)),
            pl.BlockSpec((NBLK, H), lambda i: (i, 0)),
        ],
        out_shape=[
            jax.ShapeDtypeStruct((NP, H), f32),
            jax.ShapeDtypeStruct((NP, H), f32),
        ],
    )(hp, w1a, w1b, row1(b_e1))

    mesh = plsc.VectorSubcoreMesh(core_axis_name="c", subcore_axis_name="s",
                                  num_cores=NC, num_subcores=NS)

    @functools.partial(
        pl.kernel, mesh=mesh,
        out_type=jax.ShapeDtypeStruct((EP, H), f32),
        scratch_types=[
            pltpu.VMEM((NCH, CHUNK), jnp.int32),
            pltpu.VMEM((NCH, CHUNK), jnp.int32),
            pltpu.VMEM((2, CHUNK, H), f32),
            pltpu.VMEM((2, CHUNK, H), f32),
            pltpu.SemaphoreType.DMA,
            pltpu.SemaphoreType.DMA,
            pltpu.SemaphoreType.DMA,
            pltpu.SemaphoreType.DMA,
        ],
    )
    def gather_pq(p_hbm, q_hbm, ei_hbm, x_hbm,
                  idxr, idxc, bufp, bufq, gs0, gs1, ws0, ws1):
        cid = lax.axis_index("c")
        sid = lax.axis_index("s")
        wid = sid * NC + cid
        base = wid * EW
        pltpu.sync_copy(ei_hbm.at[0, wid], idxr)
        pltpu.sync_copy(ei_hbm.at[1, wid], idxc)
        gs = (gs0, gs1)
        ws = (ws0, ws1)

        def fire_gather(j, b):
            pltpu.async_copy(p_hbm.at[idxr.at[j]], bufp.at[b], gs[b])
            pltpu.async_copy(q_hbm.at[idxc.at[j]], bufq.at[b], gs[b])

        def wait_gather(b):
            pltpu.make_async_copy(p_hbm.at[idxr.at[0]], bufp.at[b], gs[b]).wait()
            pltpu.make_async_copy(q_hbm.at[idxc.at[0]], bufq.at[b], gs[b]).wait()

        def fire_write(j, b):
            off = base + j * CHUNK
            pltpu.async_copy(bufp.at[b], x_hbm.at[pl.ds(off, CHUNK)], ws[b])

        def wait_write(b):
            pltpu.make_async_copy(bufp.at[b], x_hbm.at[pl.ds(0, CHUNK)], ws[b]).wait()

        fire_gather(0, 0)

        def step(j, b):
            wait_gather(b)

            @pl.when(j + 1 < NCH)
            def _():
                @pl.when(j >= 1)
                def _():
                    wait_write(1 - b)
                fire_gather(j + 1, 1 - b)

            def addrow(r, carry2):
                for g in range(H // 16):
                    sl = pl.ds(g * 16, 16)
                    bufp[b, r, sl] = bufp[b, r, sl] + bufq[b, r, sl]
                return carry2

            lax.fori_loop(0, CHUNK, addrow, 0)
            fire_write(j, b)

        def body(j, carry):
            @pl.when(j % 2 == 0)
            def _():
                step(j, 0)

            @pl.when(j % 2 == 1)
            def _():
                step(j, 1)

            return carry

        lax.fori_loop(0, NCH, body, 0)
        wait_write(0)
        wait_write(1)

    X = gather_pq(P, Q, ei32)

    @functools.partial(
        pl.kernel, mesh=mesh,
        out_type=(
            jax.ShapeDtypeStruct((EP, CP), f32),
            jax.ShapeDtypeStruct((EPB, 1, EBLK), f32),
        ),
        scratch_types=[
            pltpu.VMEM((NCH, CHUNK), jnp.int32),
            pltpu.VMEM((NCH, CHUNK), jnp.int32),
            pltpu.VMEM((2, CHUNK, CP), f32),
            pltpu.VMEM((2, CHUNK, CP), f32),
            pltpu.VMEM((1, CHUNK), f32),
            pltpu.SemaphoreType.DMA,
            pltpu.SemaphoreType.DMA,
            pltpu.SemaphoreType.DMA,
            pltpu.SemaphoreType.DMA,
        ],
        compiler_params=pltpu.CompilerParams(use_tc_tiling_on_sc=False,
                                             needs_layout_passes=False),
    )
    def gather_cd(cp_hbm, ei_hbm, cd_hbm, rad_hbm,
                  idxr, idxc, bufa, bufb, radb, gs0, gs1, ws0, ws1):
        cid = lax.axis_index("c")
        sid = lax.axis_index("s")
        wid = sid * NC + cid
        base = wid * EW
        pltpu.sync_copy(ei_hbm.at[0, wid], idxr)
        pltpu.sync_copy(ei_hbm.at[1, wid], idxc)
        lanes = lax.iota(jnp.int32, 16)
        gs = (gs0, gs1)
        ws = (ws0, ws1)

        def fire_gather(j, b):
            pltpu.async_copy(cp_hbm.at[idxr.at[j]], bufa.at[b], gs[b])
            pltpu.async_copy(cp_hbm.at[idxc.at[j]], bufb.at[b], gs[b])

        def wait_gather(b):
            pltpu.make_async_copy(cp_hbm.at[idxr.at[0]], bufa.at[b], gs[b]).wait()
            pltpu.make_async_copy(cp_hbm.at[idxc.at[0]], bufb.at[b], gs[b]).wait()

        def wait_write(b):
            pltpu.make_async_copy(
                bufa.at[b], cd_hbm.at[pl.ds(0, CHUNK)], ws[b]).wait()

        fire_gather(0, 0)

        def step(j, b):
            wait_gather(b)

            @pl.when(j + 1 < NCH)
            def _():
                @pl.when(j >= 1)
                def _():
                    wait_write(1 - b)
                fire_gather(j + 1, 1 - b)

            def group(k, carry2):
                acc = jnp.zeros((16,), f32)
                for u in range(16):
                    r = k * 16 + u
                    v = bufa[b, r, :] - bufb[b, r, :]
                    bufa[b, r, :] = v
                    s = jnp.sum(v * v)
                    acc = jnp.where(lanes == u, s, acc)
                radb[0, pl.ds(k * 16, 16)] = acc
                return carry2

            lax.fori_loop(0, CHUNK // 16, group, 0)
            off = base + j * CHUNK
            pltpu.async_copy(bufa.at[b], cd_hbm.at[pl.ds(off, CHUNK)], ws[b])
            pltpu.sync_copy(
                radb,
                rad_hbm.at[off // EBLK, pl.ds(0, 1), pl.ds(off % EBLK, CHUNK)])

        def chunk(j, carry):
            @pl.when(j % 2 == 0)
            def _():
                step(j, 0)

            @pl.when(j % 2 == 1)
            def _():
                step(j, 1)

            return carry

        lax.fori_loop(0, NCH, chunk, 0)
        wait_write(0)
        wait_write(1)

    CD, RAD = gather_cd(coordp, ei32)

    def edge_body(x_ref, rad_ref, ea_ref,
                  w1c_ref, w1d_ref, g1_ref, bb1_ref,
                  we2_ref, b2_ref, g2_ref, bb2_ref,
                  wc1_ref, bc1_ref, gc_ref, bbc_ref, wc2_ref,
                  ef_ref, ts_ref):
        radial = jnp.swapaxes(rad_ref[0], 0, 1)
        x1 = (x_ref[...] + radial * w1c_ref[...]
              + jnp.dot(ea_ref[...], w1d_ref[...].astype(bf16),
                        preferred_element_type=f32))
        h1 = _silu(_ln_rows(x1, g1_ref[...], bb1_ref[...]))
        x2 = jnp.dot(h1.astype(bf16), we2_ref[...].astype(bf16),
                     preferred_element_type=f32) + b2_ref[...]
        ef = _silu(_ln_rows(x2, g2_ref[...], bb2_ref[...]))
        ef_ref[...] = ef
        x3 = jnp.dot(ef.astype(bf16), wc1_ref[...].astype(bf16),
                     preferred_element_type=f32) + bc1_ref[...]
        c1 = _silu(_ln_rows(x3, gc_ref[...], bbc_ref[...]))
        s = jnp.dot(c1, wc2_ref[...], preferred_element_type=f32)
        t = s / (jnp.sqrt(radial + 1e-08) + 1.0)
        ts_ref[...] = jnp.swapaxes(t, 0, 1).reshape(1, 1, EBLK)

    big = lambda: pl.BlockSpec((EBLK, H), lambda i: (i, 0))
    wfull = lambda: pl.BlockSpec((H, H), lambda i: (0, 0))
    prow = lambda: pl.BlockSpec((1, H), lambda i: (0, 0))

    EF, TS = pl.pallas_call(
        edge_body,
        grid=(EP // EBLK,),
        in_specs=[
            big(),
            pl.BlockSpec((1, 1, EBLK), lambda i: (i, 0, 0)),
            pl.BlockSpec((EBLK, ED), lambda i: (i, 0)),
            prow(), pl.BlockSpec((ED, H), lambda i: (0, 0)), prow(), prow(),
            wfull(), prow(), prow(), prow(),
            wfull(), prow(), prow(), prow(),
            pl.BlockSpec((H, 1), lambda i: (0, 0)),
        ],
        out_specs=[big(), pl.BlockSpec((1, 1, EBLK), lambda i: (i, 0, 0))],
        out_shape=[
            jax.ShapeDtypeStruct((EP, H), f32),
            jax.ShapeDtypeStruct((EPB, 1, EBLK), f32),
        ],
    )(X, RAD, eap,
      w1c, w1d, row1(g_e1), row1(be_e1),
      W_e2, row1(b_e2), row1(g_e2), row1(be_e2),
      W_c1, row1(b_c1), row1(g_c1), row1(be_c1), W_c2)

    @functools.partial(
        pl.kernel, mesh=mesh,
        out_type=jax.ShapeDtypeStruct((NC, NP, H), f32),
        scratch_types=[
            pltpu.VMEM((NCH4, CHUNK), jnp.int32),
            pltpu.VMEM((2, CHUNK, H), f32),
            pltpu.VMEM_SHARED((NP, H), f32),
            pltpu.SemaphoreType.DMA,
            pltpu.SemaphoreType.DMA,
        ],
    )
    def scatter_n(ef_hbm, row_hbm, aggn_hbm, idx, buf, acc, ls0, ls1):
        cid = lax.axis_index("c")
        sid = lax.axis_index("s")
        zero16 = jnp.zeros((16,), f32)
        ls = (ls0, ls1)

        def zrow(r, carry):
            for g in range(H // 16):
                buf[0, r, pl.ds(g * 16, 16)] = zero16
            return carry

        lax.fori_loop(0, CHUNK, zrow, 0)
        for k in range(NZT // CHUNK):
            pltpu.sync_copy(buf.at[0], acc.at[pl.ds(sid * NZT + k * CHUNK, CHUNK)])
        plsc.subcore_barrier()

        pltpu.sync_copy(row_hbm.at[cid, sid], idx)
        base = cid * EH + sid * (EH // NS)

        def fire_load(j, b):
            off = base + j * CHUNK
            pltpu.async_copy(ef_hbm.at[pl.ds(off, CHUNK)], buf.at[b], ls[b])

        def wait_load(b):
            pltpu.make_async_copy(
                ef_hbm.at[pl.ds(0, CHUNK)], buf.at[b], ls[b]).wait()

        fire_load(0, 0)

        def step(j, b):
            wait_load(b)

            @pl.when(j + 1 < NCH4)
            def _():
                fire_load(j + 1, 1 - b)

            pltpu.sync_copy(buf.at[b], acc.at[idx.at[j]], add=True)

        def chunk(j, carry):
            @pl.when(j % 2 == 0)
            def _():
                step(j, 0)

            @pl.when(j % 2 == 1)
            def _():
                step(j, 1)

            return carry

        lax.fori_loop(0, NCH4, chunk, 0)
        plsc.subcore_barrier()
        sl = pl.ds(sid * NZT, NZT)
        pltpu.sync_copy(acc.at[sl], aggn_hbm.at[cid, sl])

    AGGN = scatter_n(EF, row4)

    @functools.partial(
        pl.kernel, mesh=mesh,
        out_type=jax.ShapeDtypeStruct((NC, NP, CP), f32),
        scratch_types=[
            pltpu.VMEM((NCH4, CHUNK), jnp.int32),
            pltpu.VMEM((2, CHUNK, CP), f32),
            pltpu.VMEM((2, 1, CHUNK), f32),
            pltpu.VMEM_SHARED((NP, CP), f32),
            pltpu.SemaphoreType.DMA,
            pltpu.SemaphoreType.DMA,
        ],
        compiler_params=pltpu.CompilerParams(use_tc_tiling_on_sc=False,
                                             needs_layout_passes=False),
    )
    def scatter_c(cd_hbm, ts_hbm, row_hbm, aggc_hbm, idx, buf, tsb, acc, ls0, ls1):
        cid = lax.axis_index("c")
        sid = lax.axis_index("s")
        zero16 = jnp.zeros((16,), f32)
        ls = (ls0, ls1)

        def zrow(r, carry):
            buf[0, r, :] = zero16
            return carry

        lax.fori_loop(0, CHUNK, zrow, 0)
        for k in range(NZT // CHUNK):
            pltpu.sync_copy(buf.at[0], acc.at[pl.ds(sid * NZT + k * CHUNK, CHUNK)])
        plsc.subcore_barrier()

        pltpu.sync_copy(row_hbm.at[cid, sid], idx)
        base = cid * EH + sid * (EH // NS)
        lanes = lax.iota(jnp.int32, 16)

        def fire_load(j, b):
            off = base + j * CHUNK
            pltpu.async_copy(cd_hbm.at[pl.ds(off, CHUNK)], buf.at[b], ls[b])
            pltpu.async_copy(
                ts_hbm.at[off // EBLK, pl.ds(0, 1), pl.ds(off % EBLK, CHUNK)],
                tsb.at[b], ls[b])

        def wait_load(b):
            pltpu.make_async_copy(
                cd_hbm.at[pl.ds(0, CHUNK)], buf.at[b], ls[b]).wait()
            pltpu.make_async_copy(
                ts_hbm.at[0, pl.ds(0, 1), pl.ds(0, CHUNK)], tsb.at[b], ls[b]).wait()

        fire_load(0, 0)

        def step(j, b):
            wait_load(b)

            @pl.when(j + 1 < NCH4)
            def _():
                fire_load(j + 1, 1 - b)

            def mul(k, carry2):
                tv = tsb[b, 0, pl.ds(k * 16, 16)]
                for u in range(16):
                    r = k * 16 + u
                    t = jnp.sum(jnp.where(lanes == u, tv, 0.0))
                    buf[b, r, :] = buf[b, r, :] * t
                return carry2

            lax.fori_loop(0, CHUNK // 16, mul, 0)
            pltpu.sync_copy(buf.at[b], acc.at[idx.at[j]], add=True)

        def chunk(j, carry):
            @pl.when(j % 2 == 0)
            def _():
                step(j, 0)

            @pl.when(j % 2 == 1)
            def _():
                step(j, 1)

            return carry

        lax.fori_loop(0, NCH4, chunk, 0)
        plsc.subcore_barrier()
        sl = pl.ds(sid * NZT, NZT)
        pltpu.sync_copy(acc.at[sl], aggc_hbm.at[cid, sl])

    AGGC = scatter_c(CD, TS, row4)

    def node_body(h_ref, co_ref, an_ref, ac_ref,
                  wa_ref, wb_ref, b1_ref, g1_ref, bb1_ref,
                  w2_ref, b2_ref, out_ref, cout_ref):
        hh = h_ref[...]
        agg = an_ref[0] + an_ref[1]
        x = (jnp.dot(hh, wa_ref[...], preferred_element_type=f32)
             + jnp.dot(agg, wb_ref[...], preferred_element_type=f32)
             + b1_ref[...])
        nn = _silu(_ln_rows(x, g1_ref[...], bb1_ref[...]))
        out_ref[...] = hh + jnp.dot(nn, w2_ref[...], preferred_element_type=f32) + b2_ref[...]
        cout_ref[...] = co_ref[...] + ac_ref[0][:, 0:3] + ac_ref[1][:, 0:3]

    out, coord_out = pl.pallas_call(
        node_body,
        grid=(N // NBLK5,),
        in_specs=[
            pl.BlockSpec((NBLK5, H), lambda i: (i, 0)),
            pl.BlockSpec((NBLK5, 3), lambda i: (i, 0)),
            pl.BlockSpec((NC, NBLK5, H), lambda i: (0, i, 0)),
            pl.BlockSpec((NC, NBLK5, CP), lambda i: (0, i, 0)),
            pl.BlockSpec((H, H), lambda i: (0, 0)),
            pl.BlockSpec((H, H), lambda i: (0, 0)),
            pl.BlockSpec((1, H), lambda i: (0, 0)),
            pl.BlockSpec((1, H), lambda i: (0, 0)),
            pl.BlockSpec((1, H), lambda i: (0, 0)),
            pl.BlockSpec((H, H), lambda i: (0, 0)),
            pl.BlockSpec((1, H), lambda i: (0, 0)),
        ],
        out_specs=[
            pl.BlockSpec((NBLK5, H), lambda i: (i, 0)),
            pl.BlockSpec((NBLK5, 3), lambda i: (i, 0)),
        ],
        out_shape=[
            jax.ShapeDtypeStruct((N, H), f32),
            jax.ShapeDtypeStruct((N, 3), f32),
        ],
    )(h, coord, AGGN, AGGC,
      wn1a, wn1b, row1(b_n1), row1(g_n1), row1(be_n1),
      W_n2, row1(b_n2))

    return (out, coord_out)


# speedup vs baseline: 4.9567x; 1.0002x over previous
import functools

import jax
import jax.numpy as jnp
from jax import lax
from jax.experimental import pallas as pl
from jax.experimental.pallas import tpu as pltpu
from jax.experimental.pallas import tpu_sc as plsc

NC = 2
NS = 16
NW = NC * NS
CHUNK = 128
CP = 16


def _pick_div(n, cap, mult=1):
    for d in range(min(n, cap), 0, -1):
        if n % d == 0 and d % mult == 0:
            return d
    return 1


def _ln_rows(x, g, b):
    hh = x.shape[-1]
    ones_col = jnp.ones((hh, 1), jnp.float32)
    m = jnp.dot(x, ones_col, preferred_element_type=jnp.float32) * (1.0 / hh)
    s2 = jnp.dot(x * x, ones_col, preferred_element_type=jnp.float32) * (1.0 / hh)
    v = s2 - m * m
    rstd = lax.rsqrt(v + 1e-5)
    return (x - m) * rstd * g + b


def _silu(x):
    return 0.5 * x * (1.0 + jnp.tanh(0.5 * x))


def kernel(h, edge_index, coord, edge_attr, W_e1, b_e1, g_e1, be_e1, W_e2,
           b_e2, g_e2, be_e2, W_n1, b_n1, g_n1, be_n1, W_n2, b_n2, W_c1,
           b_c1, g_c1, be_c1, W_c2):
    N, H = h.shape
    E = edge_index.shape[1]
    f32 = jnp.float32
    bf16 = jnp.bfloat16

    NCH = -(-E // (NW * CHUNK))
    NCH += NCH % 2
    EP = NW * NCH * CHUNK
    EW = EP // NW
    EH = EP // NC
    NCH4 = EH // NS // CHUNK
    NP = -(-(N + 1) // 1024) * 1024
    NZT = NP // NS
    NBLK = _pick_div(NP, 1024, 8)
    NBLK5 = _pick_div(N, 1024, 8)
    EBLK = _pick_div(EP, 2048, 8)
    EPB = EP // EBLK

    row1 = lambda a: a.reshape(1, H)

    hp = jnp.pad(h, ((0, NP - N), (0, 0)))
    coordp = jnp.pad(coord, ((0, NP - N), (0, CP - coord.shape[1])))
    pad_idx = N + jnp.arange(EP - E, dtype=edge_index.dtype) % (NP - N)
    eip = jnp.concatenate(
        [edge_index, jnp.broadcast_to(pad_idx, (2, EP - E))], axis=1)
    ei32 = eip.reshape(2, NW, NCH, CHUNK)
    row4 = eip[0].reshape(NC, NS, NCH4, CHUNK)
    eap = jnp.pad(edge_attr, ((0, EP - E), (0, 0))).astype(bf16)
    w1a = W_e1[0:H]
    w1b = W_e1[H:2 * H]
    w1c = W_e1[2 * H:2 * H + 1]
    w1d = W_e1[2 * H + 1:]
    ED = w1d.shape[0]
    wn1a = W_n1[0:H]
    wn1b = W_n1[H:2 * H]

    def pq_body(h_ref, wa_ref, wb_ref, b1_ref, p_ref, q_ref):
        hh = h_ref[...]
        p_ref[...] = jnp.dot(hh, wa_ref[...], preferred_element_type=f32) + b1_ref[...]
        q_ref[...] = jnp.dot(hh, wb_ref[...], preferred_element_type=f32)

    P, Q = pl.pallas_call(
        pq_body,
        grid=(NP // NBLK,),
        in_specs=[
            pl.BlockSpec((NBLK, H), lambda i: (i, 0)),
            pl.BlockSpec((H, H), lambda i: (0, 0)),
            pl.BlockSpec((H, H), lambda i: (0, 0)),
            pl.BlockSpec((1, H), lambda i: (0, 0)),
        ],
        out_specs=[
            pl.BlockSpec((NBLK, H), lambda i: (i, 0)),
            pl.BlockSpec((NBLK, H), lambda i: (i, 0)),
        ],
        out_shape=[
            jax.ShapeDtypeStruct((NP, H), f32),
            jax.ShapeDtypeStruct((NP, H), f32),
        ],
    )(hp, w1a, w1b, row1(b_e1))

    mesh = plsc.VectorSubcoreMesh(core_axis_name="c", subcore_axis_name="s",
                                  num_cores=NC, num_subcores=NS)

    @functools.partial(
        pl.kernel, mesh=mesh,
        out_type=jax.ShapeDtypeStruct((EP, H), f32),
        scratch_types=[
            pltpu.VMEM((NCH, CHUNK), jnp.int32),
            pltpu.VMEM((NCH, CHUNK), jnp.int32),
            pltpu.VMEM((2, CHUNK, H), f32),
            pltpu.VMEM((2, CHUNK, H), f32),
            pltpu.SemaphoreType.DMA,
            pltpu.SemaphoreType.DMA,
            pltpu.SemaphoreType.DMA,
            pltpu.SemaphoreType.DMA,
        ],
    )
    def gather_pq(p_hbm, q_hbm, ei_hbm, x_hbm,
                  idxr, idxc, bufp, bufq, gs0, gs1, ws0, ws1):
        cid = lax.axis_index("c")
        sid = lax.axis_index("s")
        wid = sid * NC + cid
        base = wid * EW
        pltpu.sync_copy(ei_hbm.at[0, wid], idxr)
        pltpu.sync_copy(ei_hbm.at[1, wid], idxc)
        gs = (gs0, gs1)
        ws = (ws0, ws1)

        def fire_gather(j, b):
            pltpu.async_copy(p_hbm.at[idxr.at[j]], bufp.at[b], gs[b])
            pltpu.async_copy(q_hbm.at[idxc.at[j]], bufq.at[b], gs[b])

        def wait_gather(b):
            pltpu.make_async_copy(p_hbm.at[idxr.at[0]], bufp.at[b], gs[b]).wait()
            pltpu.make_async_copy(q_hbm.at[idxc.at[0]], bufq.at[b], gs[b]).wait()

        def fire_write(j, b):
            off = base + j * CHUNK
            pltpu.async_copy(bufp.at[b], x_hbm.at[pl.ds(off, CHUNK)], ws[b])

        def wait_write(b):
            pltpu.make_async_copy(bufp.at[b], x_hbm.at[pl.ds(0, CHUNK)], ws[b]).wait()

        fire_gather(0, 0)

        def step(j, b):
            wait_gather(b)

            @pl.when(j + 1 < NCH)
            def _():
                @pl.when(j >= 1)
                def _():
                    wait_write(1 - b)
                fire_gather(j + 1, 1 - b)

            def addrow(r, carry2):
                for g in range(H // 16):
                    sl = pl.ds(g * 16, 16)
                    bufp[b, r, sl] = bufp[b, r, sl] + bufq[b, r, sl]
                return carry2

            lax.fori_loop(0, CHUNK, addrow, 0)
            fire_write(j, b)

        def body(j, carry):
            @pl.when(j % 2 == 0)
            def _():
                step(j, 0)

            @pl.when(j % 2 == 1)
            def _():
                step(j, 1)

            return carry

        lax.fori_loop(0, NCH, body, 0)
        wait_write(0)
        wait_write(1)

    X = gather_pq(P, Q, ei32)

    @functools.partial(
        pl.kernel, mesh=mesh,
        out_type=(
            jax.ShapeDtypeStruct((EP, CP), f32),
            jax.ShapeDtypeStruct((EPB, 1, EBLK), f32),
        ),
        scratch_types=[
            pltpu.VMEM((NCH, CHUNK), jnp.int32),
            pltpu.VMEM((NCH, CHUNK), jnp.int32),
            pltpu.VMEM((2, CHUNK, CP), f32),
            pltpu.VMEM((2, CHUNK, CP), f32),
            pltpu.VMEM((1, CHUNK), f32),
            pltpu.SemaphoreType.DMA,
            pltpu.SemaphoreType.DMA,
            pltpu.SemaphoreType.DMA,
            pltpu.SemaphoreType.DMA,
        ],
        compiler_params=pltpu.CompilerParams(use_tc_tiling_on_sc=False,
                                             needs_layout_passes=False),
    )
    def gather_cd(cp_hbm, ei_hbm, cd_hbm, rad_hbm,
                  idxr, idxc, bufa, bufb, radb, gs0, gs1, ws0, ws1):
        cid = lax.axis_index("c")
        sid = lax.axis_index("s")
        wid = sid * NC + cid
        base = wid * EW
        pltpu.sync_copy(ei_hbm.at[0, wid], idxr)
        pltpu.sync_copy(ei_hbm.at[1, wid], idxc)
        lanes = lax.iota(jnp.int32, 16)
        gs = (gs0, gs1)
        ws = (ws0, ws1)

        def fire_gather(j, b):
            pltpu.async_copy(cp_hbm.at[idxr.at[j]], bufa.at[b], gs[b])
            pltpu.async_copy(cp_hbm.at[idxc.at[j]], bufb.at[b], gs[b])

        def wait_gather(b):
            pltpu.make_async_copy(cp_hbm.at[idxr.at[0]], bufa.at[b], gs[b]).wait()
            pltpu.make_async_copy(cp_hbm.at[idxc.at[0]], bufb.at[b], gs[b]).wait()

        def wait_write(b):
            pltpu.make_async_copy(
                bufa.at[b], cd_hbm.at[pl.ds(0, CHUNK)], ws[b]).wait()

        fire_gather(0, 0)

        def step(j, b):
            wait_gather(b)

            @pl.when(j + 1 < NCH)
            def _():
                @pl.when(j >= 1)
                def _():
                    wait_write(1 - b)
                fire_gather(j + 1, 1 - b)

            def group(k, carry2):
                acc = jnp.zeros((16,), f32)
                for u in range(16):
                    r = k * 16 + u
                    v = bufa[b, r, :] - bufb[b, r, :]
                    bufa[b, r, :] = v
                    s = jnp.sum(v * v)
                    acc = jnp.where(lanes == u, s, acc)
                radb[0, pl.ds(k * 16, 16)] = acc
                return carry2

            lax.fori_loop(0, CHUNK // 16, group, 0)
            off = base + j * CHUNK
            pltpu.async_copy(bufa.at[b], cd_hbm.at[pl.ds(off, CHUNK)], ws[b])
            pltpu.sync_copy(
                radb,
                rad_hbm.at[off // EBLK, pl.ds(0, 1), pl.ds(off % EBLK, CHUNK)])

        def chunk(j, carry):
            @pl.when(j % 2 == 0)
            def _():
                step(j, 0)

            @pl.when(j % 2 == 1)
            def _():
                step(j, 1)

            return carry

        lax.fori_loop(0, NCH, chunk, 0)
        wait_write(0)
        wait_write(1)

    CD, RAD = gather_cd(coordp, ei32)

    def edge_body(x_ref, rad_ref, ea_ref,
                  w1c_ref, w1d_ref, g1_ref, bb1_ref,
                  we2_ref, b2_ref, g2_ref, bb2_ref,
                  wc1_ref, bc1_ref, gc_ref, bbc_ref, wc2_ref,
                  ef_ref, ts_ref):
        radial = jnp.swapaxes(rad_ref[0], 0, 1)
        x1 = (x_ref[...] + radial * w1c_ref[...]
              + jnp.dot(ea_ref[...], w1d_ref[...].astype(bf16),
                        preferred_element_type=f32))
        h1 = _silu(_ln_rows(x1, g1_ref[...], bb1_ref[...]))
        x2 = jnp.dot(h1.astype(bf16), we2_ref[...].astype(bf16),
                     preferred_element_type=f32) + b2_ref[...]
        ef = _silu(_ln_rows(x2, g2_ref[...], bb2_ref[...]))
        ef_ref[...] = ef
        x3 = jnp.dot(ef.astype(bf16), wc1_ref[...].astype(bf16),
                     preferred_element_type=f32) + bc1_ref[...]
        c1 = _silu(_ln_rows(x3, gc_ref[...], bbc_ref[...]))
        s = jnp.dot(c1, wc2_ref[...], preferred_element_type=f32)
        t = s / (jnp.sqrt(radial + 1e-08) + 1.0)
        ts_ref[...] = jnp.swapaxes(t, 0, 1).reshape(1, 1, EBLK)

    big = lambda: pl.BlockSpec((EBLK, H), lambda i: (i, 0))
    wfull = lambda: pl.BlockSpec((H, H), lambda i: (0, 0))
    prow = lambda: pl.BlockSpec((1, H), lambda i: (0, 0))

    EF, TS = pl.pallas_call(
        edge_body,
        grid=(EP // EBLK,),
        in_specs=[
            big(),
            pl.BlockSpec((1, 1, EBLK), lambda i: (i, 0, 0)),
            pl.BlockSpec((EBLK, ED), lambda i: (i, 0)),
            prow(), pl.BlockSpec((ED, H), lambda i: (0, 0)), prow(), prow(),
            wfull(), prow(), prow(), prow(),
            wfull(), prow(), prow(), prow(),
            pl.BlockSpec((H, 1), lambda i: (0, 0)),
        ],
        out_specs=[big(), pl.BlockSpec((1, 1, EBLK), lambda i: (i, 0, 0))],
        out_shape=[
            jax.ShapeDtypeStruct((EP, H), f32),
            jax.ShapeDtypeStruct((EPB, 1, EBLK), f32),
        ],
    )(X, RAD, eap,
      w1c, w1d, row1(g_e1), row1(be_e1),
      W_e2, row1(b_e2), row1(g_e2), row1(be_e2),
      W_c1, row1(b_c1), row1(g_c1), row1(be_c1), W_c2)

    @functools.partial(
        pl.kernel, mesh=mesh,
        out_type=jax.ShapeDtypeStruct((NC, NP, H), f32),
        scratch_types=[
            pltpu.VMEM((NCH4, CHUNK), jnp.int32),
            pltpu.VMEM((2, CHUNK, H), f32),
            pltpu.VMEM_SHARED((NP, H), f32),
            pltpu.SemaphoreType.DMA,
            pltpu.SemaphoreType.DMA,
        ],
    )
    def scatter_n(ef_hbm, row_hbm, aggn_hbm, idx, buf, acc, ls0, ls1):
        cid = lax.axis_index("c")
        sid = lax.axis_index("s")
        zero16 = jnp.zeros((16,), f32)
        ls = (ls0, ls1)

        def zrow(r, carry):
            for g in range(H // 16):
                buf[0, r, pl.ds(g * 16, 16)] = zero16
            return carry

        lax.fori_loop(0, CHUNK, zrow, 0)
        for k in range(NZT // CHUNK):
            pltpu.sync_copy(buf.at[0], acc.at[pl.ds(sid * NZT + k * CHUNK, CHUNK)])
        plsc.subcore_barrier()

        pltpu.sync_copy(row_hbm.at[cid, sid], idx)
        base = cid * EH + sid * (EH // NS)

        def fire_load(j, b):
            off = base + j * CHUNK
            pltpu.async_copy(ef_hbm.at[pl.ds(off, CHUNK)], buf.at[b], ls[b])

        def wait_load(b):
            pltpu.make_async_copy(
                ef_hbm.at[pl.ds(0, CHUNK)], buf.at[b], ls[b]).wait()

        fire_load(0, 0)

        def step(j, b):
            wait_load(b)

            @pl.when(j + 1 < NCH4)
            def _():
                fire_load(j + 1, 1 - b)

            pltpu.sync_copy(buf.at[b], acc.at[idx.at[j]], add=True)

        def chunk(j, carry):
            @pl.when(j % 2 == 0)
            def _():
                step(j, 0)

            @pl.when(j % 2 == 1)
            def _():
                step(j, 1)

            return carry

        lax.fori_loop(0, NCH4, chunk, 0)
        plsc.subcore_barrier()
        sl = pl.ds(sid * NZT, NZT)
        pltpu.sync_copy(acc.at[sl], aggn_hbm.at[cid, sl])

    AGGN = scatter_n(EF, row4)

    @functools.partial(
        pl.kernel, mesh=mesh,
        out_type=jax.ShapeDtypeStruct((NC, NP, CP), f32),
        scratch_types=[
            pltpu.VMEM((NCH4, CHUNK), jnp.int32),
            pltpu.VMEM((2, CHUNK, CP), f32),
            pltpu.VMEM((2, 1, CHUNK), f32),
            pltpu.VMEM_SHARED((NP, CP), f32),
            pltpu.SemaphoreType.DMA,
            pltpu.SemaphoreType.DMA,
        ],
        compiler_params=pltpu.CompilerParams(use_tc_tiling_on_sc=False,
                                             needs_layout_passes=False),
    )
    def scatter_c(cd_hbm, ts_hbm, row_hbm, aggc_hbm, idx, buf, tsb, acc, ls0, ls1):
        cid = lax.axis_index("c")
        sid = lax.axis_index("s")
        zero16 = jnp.zeros((16,), f32)
        ls = (ls0, ls1)

        def zrow(r, carry):
            buf[0, r, :] = zero16
            return carry

        lax.fori_loop(0, CHUNK, zrow, 0)
        for k in range(NZT // CHUNK):
            pltpu.sync_copy(buf.at[0], acc.at[pl.ds(sid * NZT + k * CHUNK, CHUNK)])
        plsc.subcore_barrier()

        pltpu.sync_copy(row_hbm.at[cid, sid], idx)
        base = cid * EH + sid * (EH // NS)
        lanes = lax.iota(jnp.int32, 16)

        def fire_load(j, b):
            off = base + j * CHUNK
            pltpu.async_copy(cd_hbm.at[pl.ds(off, CHUNK)], buf.at[b], ls[b])
            pltpu.async_copy(
                ts_hbm.at[off // EBLK, pl.ds(0, 1), pl.ds(off % EBLK, CHUNK)],
                tsb.at[b], ls[b])

        def wait_load(b):
            pltpu.make_async_copy(
                cd_hbm.at[pl.ds(0, CHUNK)], buf.at[b], ls[b]).wait()
            pltpu.make_async_copy(
                ts_hbm.at[0, pl.ds(0, 1), pl.ds(0, CHUNK)], tsb.at[b], ls[b]).wait()

        fire_load(0, 0)

        def step(j, b):
            wait_load(b)

            @pl.when(j + 1 < NCH4)
            def _():
                fire_load(j + 1, 1 - b)

            def mul(k, carry2):
                tv = tsb[b, 0, pl.ds(k * 16, 16)]
                for u in range(16):
                    r = k * 16 + u
                    t = jnp.sum(jnp.where(lanes == u, tv, 0.0))
                    buf[b, r, :] = buf[b, r, :] * t
                return carry2

            lax.fori_loop(0, CHUNK // 16, mul, 0)
            pltpu.sync_copy(buf.at[b], acc.at[idx.at[j]], add=True)

        def chunk(j, carry):
            @pl.when(j % 2 == 0)
            def _():
                step(j, 0)

            @pl.when(j % 2 == 1)
            def _():
                step(j, 1)

            return carry

        lax.fori_loop(0, NCH4, chunk, 0)
        plsc.subcore_barrier()
        sl = pl.ds(sid * NZT, NZT)
        pltpu.sync_copy(acc.at[sl], aggc_hbm.at[cid, sl])

    AGGC = scatter_c(CD, TS, row4)

    def node_body(h_ref, co_ref, an_ref, ac_ref,
                  wa_ref, wb_ref, b1_ref, g1_ref, bb1_ref,
                  w2_ref, b2_ref, out_ref, cout_ref):
        hh = h_ref[...]
        agg = an_ref[0] + an_ref[1]
        x = (jnp.dot(hh, wa_ref[...], preferred_element_type=f32)
             + jnp.dot(agg, wb_ref[...], preferred_element_type=f32)
             + b1_ref[...])
        nn = _silu(_ln_rows(x, g1_ref[...], bb1_ref[...]))
        out_ref[...] = hh + jnp.dot(nn, w2_ref[...], preferred_element_type=f32) + b2_ref[...]
        cout_ref[...] = co_ref[...] + ac_ref[0][:, 0:3] + ac_ref[1][:, 0:3]

    out, coord_out = pl.pallas_call(
        node_body,
        grid=(N // NBLK5,),
        in_specs=[
            pl.BlockSpec((NBLK5, H), lambda i: (i, 0)),
            pl.BlockSpec((NBLK5, 3), lambda i: (i, 0)),
            pl.BlockSpec((NC, NBLK5, H), lambda i: (0, i, 0)),
            pl.BlockSpec((NC, NBLK5, CP), lambda i: (0, i, 0)),
            pl.BlockSpec((H, H), lambda i: (0, 0)),
            pl.BlockSpec((H, H), lambda i: (0, 0)),
            pl.BlockSpec((1, H), lambda i: (0, 0)),
            pl.BlockSpec((1, H), lambda i: (0, 0)),
            pl.BlockSpec((1, H), lambda i: (0, 0)),
            pl.BlockSpec((H, H), lambda i: (0, 0)),
            pl.BlockSpec((1, H), lambda i: (0, 0)),
        ],
        out_specs=[
            pl.BlockSpec((NBLK5, H), lambda i: (i, 0)),
            pl.BlockSpec((NBLK5, 3), lambda i: (i, 0)),
        ],
        out_shape=[
            jax.ShapeDtypeStruct((N, H), f32),
            jax.ShapeDtypeStruct((N, 3), f32),
        ],
    )(h, coord, AGGN, AGGC,
      wn1a, wn1b, row1(b_n1), row1(g_n1), row1(be_n1),
      W_n2, row1(b_n2))

    return (out, coord_out)
